# Initial kernel scaffold; baseline (speedup 1.0000x reference)
import jax
import jax.numpy as jnp
from jax.experimental import pallas as pl


def kernel(features, W1, b1, W2, b2, W3, b3):
    raise NotImplementedError("write your pallas kernel here")


# speedup vs baseline: 8.7224x; 8.7224x over previous
import functools

import jax
import jax.numpy as jnp
from jax import lax
from jax.experimental import pallas as pl
from jax.experimental.pallas import tpu as pltpu
from jax.experimental.pallas import tpu_sc as plsc

_K = 10
_KPAD = 16
_NEG = -3.0e38

_NC = 2
_NS = 16
_NW = _NC * _NS


def _topk_body(xr_ref, xc_ref, idx_ref):
    xr = xr_ref[0]
    xc = xc_ref[0]
    blk, _ = xr.shape
    n = xc.shape[0]
    sq = jnp.sum(xc * xc, axis=1)
    s = 2.0 * lax.dot_general(xr, xc, (((1,), (1,)), ((), ())),
                              preferred_element_type=jnp.float32)
    s = s - sq[None, :]
    iota = lax.broadcasted_iota(jnp.int32, (blk, n), 1)
    cols = []
    for _ in range(_K):
        m = jnp.max(s, axis=1, keepdims=True)
        hit = s == m
        am = jnp.min(jnp.where(hit, iota, n), axis=1, keepdims=True)
        cols.append(am)
        s = jnp.where(iota == am, _NEG, s)
    self_idx = (pl.program_id(1) * blk
                + lax.broadcasted_iota(jnp.int32, (blk, 1), 0))
    cols.extend([self_idx] * (_KPAD - _K))
    base = pl.program_id(0) * n
    idx_ref[0] = jnp.concatenate(cols, axis=1) + base


def _topk(xt, blk=256):
    b, n, c = xt.shape
    return pl.pallas_call(
        _topk_body,
        grid=(b, n // blk),
        in_specs=[
            pl.BlockSpec((1, blk, c), lambda bi, i: (bi, i, 0)),
            pl.BlockSpec((1, n, c), lambda bi, i: (bi, 0, 0)),
        ],
        out_specs=pl.BlockSpec((1, blk, _KPAD), lambda bi, i: (bi, i, 0)),
        out_shape=jax.ShapeDtypeStruct((b, n, _KPAD), jnp.int32),
    )(xt, xt)


def _proj_body(x_ref, wa_ref, wb_ref, b_ref, p_ref, q_ref):
    x = x_ref[...]
    wb = wb_ref[...]
    wd = wa_ref[...] - wb
    dn = (((1,), (1,)), ((), ()))
    p_ref[...] = (lax.dot_general(x, wd, dn, preferred_element_type=jnp.float32)
                  + b_ref[...])
    q_ref[...] = lax.dot_general(x, wb, dn, preferred_element_type=jnp.float32)


def _proj(xf, wa, wb, bias, blk=512):
    m, c = xf.shape
    o = wa.shape[0]
    return pl.pallas_call(
        _proj_body,
        grid=(m // blk,),
        in_specs=[
            pl.BlockSpec((blk, c), lambda i: (i, 0)),
            pl.BlockSpec((o, c), lambda i: (0, 0)),
            pl.BlockSpec((o, c), lambda i: (0, 0)),
            pl.BlockSpec((1, o), lambda i: (0, 0)),
        ],
        out_specs=[
            pl.BlockSpec((blk, o), lambda i: (i, 0)),
            pl.BlockSpec((blk, o), lambda i: (i, 0)),
        ],
        out_shape=[
            jax.ShapeDtypeStruct((m, o), jnp.float32),
            jax.ShapeDtypeStruct((m, o), jnp.float32),
        ],
    )(xf, wa, wb, bias.reshape(1, o))


def _gather_sub(idx_flat, xf):
    m, c = xf.shape
    per_w = m // _NW
    g = 8
    iters = per_w // g
    mesh = plsc.VectorSubcoreMesh(core_axis_name="c", subcore_axis_name="s",
                                  num_cores=_NC, num_subcores=_NS)

    @functools.partial(
        pl.kernel,
        out_type=jax.ShapeDtypeStruct((_K, m, c), jnp.float32),
        mesh=mesh,
        scratch_types=[
            pltpu.VMEM((g * _K,), jnp.int32),
            pltpu.VMEM((g * _K, c), jnp.float32),
            pltpu.VMEM((g, c), jnp.float32),
            pltpu.VMEM((_K, g, c), jnp.float32),
            pltpu.SemaphoreType.DMA,
        ],
    )
    def kern(idx_hbm, x_hbm, e_hbm, idx_v, rows_v, xi_v, o3_v, sem):
        wid = lax.axis_index("c") * _NS + lax.axis_index("s")

        def body(it, carry):
            base = wid * per_w + it * g
            pltpu.sync_copy(idx_hbm.at[pl.ds(base * _K, g * _K)], idx_v)
            pltpu.async_copy(x_hbm.at[idx_v], rows_v, sem).wait()
            pltpu.sync_copy(x_hbm.at[pl.ds(base, g)], xi_v)
            for r in range(_K):
                for gg in range(g):
                    for cc in range(c // 16):
                        sl = pl.ds(cc * 16, 16)
                        o3_v[r, gg, sl] = rows_v[gg * _K + r, sl] - xi_v[gg, sl]
            for r in range(_K):
                pltpu.sync_copy(o3_v.at[r], e_hbm.at[r, pl.ds(base, g)])
            return carry

        lax.fori_loop(0, iters, body, 0)

    return kern(idx_flat, xf)


def _edge_mlp_body(e_ref, xi_ref, w_ref, b_ref, o_ref):
    e3 = e_ref[...]
    xi = xi_ref[...]
    k, blk, c = e3.shape
    xib = jnp.broadcast_to(xi[None], (k, blk, c))
    edge = jnp.concatenate([xib, e3], axis=2).reshape(k * blk, 2 * c)
    h = lax.dot_general(edge, w_ref[...], (((1,), (1,)), ((), ())),
                        preferred_element_type=jnp.float32)
    h = h + b_ref[...]
    h = jnp.where(h >= 0, h, 0.2 * h)
    o_ref[...] = jnp.max(h.reshape(k, blk, -1), axis=0)


def _edge_mlp(e3, xf, w, bias, blk=128):
    m, c = xf.shape
    o = w.shape[0]
    return pl.pallas_call(
        _edge_mlp_body,
        grid=(m // blk,),
        in_specs=[
            pl.BlockSpec((_K, blk, c), lambda i: (0, i, 0)),
            pl.BlockSpec((blk, c), lambda i: (i, 0)),
            pl.BlockSpec((o, 2 * c), lambda i: (0, 0)),
            pl.BlockSpec((1, o), lambda i: (0, 0)),
        ],
        out_specs=pl.BlockSpec((blk, o), lambda i: (i, 0)),
        out_shape=jax.ShapeDtypeStruct((m, o), jnp.float32),
    )(e3, xf, w, bias.reshape(1, o))


def _gather_max(idx_flat, q, p):
    m, c = q.shape
    per_w = m // _NW
    g = 8
    iters = per_w // g
    mesh = plsc.VectorSubcoreMesh(core_axis_name="c", subcore_axis_name="s",
                                  num_cores=_NC, num_subcores=_NS)

    @functools.partial(
        pl.kernel,
        out_type=jax.ShapeDtypeStruct((m, c), jnp.float32),
        mesh=mesh,
        scratch_types=[
            pltpu.VMEM((g * _K,), jnp.int32),
            pltpu.VMEM((g * _K, c), jnp.float32),
            pltpu.VMEM((g, c), jnp.float32),
            pltpu.VMEM((g, c), jnp.float32),
            pltpu.SemaphoreType.DMA,
        ],
    )
    def kern(idx_hbm, q_hbm, p_hbm, out_hbm, idx_v, rows_v, p_v, o_v, sem):
        wid = lax.axis_index("c") * _NS + lax.axis_index("s")

        def body(it, carry):
            base = wid * per_w + it * g
            pltpu.sync_copy(idx_hbm.at[pl.ds(base * _K, g * _K)], idx_v)
            pltpu.async_copy(q_hbm.at[idx_v], rows_v, sem).wait()
            pltpu.sync_copy(p_hbm.at[pl.ds(base, g)], p_v)
            for gg in range(g):
                for cc in range(c // 16):
                    sl = pl.ds(cc * 16, 16)
                    acc = rows_v[gg * _K, sl]
                    for r in range(1, _K):
                        acc = jnp.maximum(acc, rows_v[gg * _K + r, sl])
                    h = p_v[gg, sl] + acc
                    o_v[gg, sl] = jnp.maximum(h, 0.2 * h)
            pltpu.sync_copy(o_v, out_hbm.at[pl.ds(base, g)])
            return carry

        lax.fori_loop(0, iters, body, 0)

    return kern(idx_flat, q, p)


def _pw_body(x0_ref, x1_ref, x2_ref, wa_ref, wb_ref, wc_ref, b_ref, o_ref):
    dn = (((1,), (1,)), ((), ()))
    acc = lax.dot_general(x0_ref[...], wa_ref[...], dn,
                          preferred_element_type=jnp.float32)
    acc += lax.dot_general(x1_ref[...], wb_ref[...], dn,
                           preferred_element_type=jnp.float32)
    acc += lax.dot_general(x2_ref[...], wc_ref[...], dn,
                           preferred_element_type=jnp.float32)
    acc += b_ref[...]
    o_ref[...] = jnp.maximum(acc, 0.2 * acc)


def _pointwise(x0f, x1f, x2f, w3, b3, blk=512):
    m, c0 = x0f.shape
    c1 = x1f.shape[1]
    c2 = x2f.shape[1]
    o = w3.shape[0]
    wa = w3[:, :c0]
    wb = w3[:, c0:c0 + c1]
    wc = w3[:, c0 + c1:]
    return pl.pallas_call(
        _pw_body,
        grid=(m // blk,),
        in_specs=[
            pl.BlockSpec((blk, c0), lambda i: (i, 0)),
            pl.BlockSpec((blk, c1), lambda i: (i, 0)),
            pl.BlockSpec((blk, c2), lambda i: (i, 0)),
            pl.BlockSpec((o, c0), lambda i: (0, 0)),
            pl.BlockSpec((o, c1), lambda i: (0, 0)),
            pl.BlockSpec((o, c2), lambda i: (0, 0)),
            pl.BlockSpec((1, o), lambda i: (0, 0)),
        ],
        out_specs=pl.BlockSpec((blk, o), lambda i: (i, 0)),
        out_shape=jax.ShapeDtypeStruct((m, o), jnp.float32),
    )(x0f, x1f, x2f, wa, wb, wc, b3.reshape(1, o))


def _edgeconv(xt, w, bias):
    b, n, c = xt.shape
    idx = _topk(xt)
    idx10 = idx[:, :, :_K].reshape(-1)
    xf = xt.reshape(b * n, c)
    p, q = _proj(xf, w[:, :c], w[:, c:], bias)
    xo = _gather_max(idx10, q, p)
    return xo.reshape(b, n, -1)


def kernel(features, W1, b1, W2, b2, W3, b3):
    b, f, n = features.shape
    x0t = jnp.transpose(features, (0, 2, 1))
    x0f = x0t.reshape(b * n, f)
    idx1 = _topk(x0t)[:, :, :_K].reshape(-1)
    e3 = _gather_sub(idx1, x0f)
    x1t = _edge_mlp(e3, x0f, W1, b1).reshape(b, n, f)
    x2t = _edgeconv(x1t, W2, b2)
    m = b * n
    out = _pointwise(x0t.reshape(m, -1), x1t.reshape(m, -1),
                     x2t.reshape(m, -1), W3, b3)
    return jnp.transpose(out.reshape(b, n, f), (0, 2, 1))


# speedup vs baseline: 12.3635x; 1.4174x over previous
import functools

import jax
import jax.numpy as jnp
from jax import lax
from jax.experimental import pallas as pl
from jax.experimental.pallas import tpu as pltpu
from jax.experimental.pallas import tpu_sc as plsc

_K = 10
_KPAD = 16
_NEG = -3.0e38

_NC = 2
_NS = 16
_NW = _NC * _NS


def _topk_body(xr_ref, xc_ref, idx_ref):
    xr = xr_ref[0]
    xc = xc_ref[0]
    blk, _ = xr.shape
    n = xc.shape[0]
    sq = jnp.sum(xc * xc, axis=1)
    s = 2.0 * lax.dot_general(xr, xc, (((1,), (1,)), ((), ())),
                              preferred_element_type=jnp.float32)
    s = s - sq[None, :]
    iota = lax.broadcasted_iota(jnp.int32, (blk, n), 1)
    cols = []
    for _ in range(_K):
        m = jnp.max(s, axis=1, keepdims=True)
        hit = s == m
        am = jnp.min(jnp.where(hit, iota, n), axis=1, keepdims=True)
        cols.append(am)
        s = jnp.where(iota == am, _NEG, s)
    self_idx = (pl.program_id(1) * blk
                + lax.broadcasted_iota(jnp.int32, (blk, 1), 0))
    cols.extend([self_idx] * (_KPAD - _K))
    base = pl.program_id(0) * n
    idx_ref[0] = jnp.concatenate(cols, axis=1) + base


def _topk(xt, blk=256):
    b, n, c = xt.shape
    return pl.pallas_call(
        _topk_body,
        grid=(b, n // blk),
        in_specs=[
            pl.BlockSpec((1, blk, c), lambda bi, i: (bi, i, 0)),
            pl.BlockSpec((1, n, c), lambda bi, i: (bi, 0, 0)),
        ],
        out_specs=pl.BlockSpec((1, blk, _KPAD), lambda bi, i: (bi, i, 0)),
        out_shape=jax.ShapeDtypeStruct((b, n, _KPAD), jnp.int32),
    )(xt, xt)


def _proj_body(x_ref, wa_ref, wb_ref, b_ref, p_ref, q_ref):
    x = x_ref[...]
    wb = wb_ref[...]
    wd = wa_ref[...] - wb
    dn = (((1,), (1,)), ((), ()))
    p_ref[...] = (lax.dot_general(x, wd, dn, preferred_element_type=jnp.float32)
                  + b_ref[...])
    q_ref[...] = lax.dot_general(x, wb, dn, preferred_element_type=jnp.float32)


def _proj(xf, wa, wb, bias, blk=512):
    m, c = xf.shape
    o = wa.shape[0]
    return pl.pallas_call(
        _proj_body,
        grid=(m // blk,),
        in_specs=[
            pl.BlockSpec((blk, c), lambda i: (i, 0)),
            pl.BlockSpec((o, c), lambda i: (0, 0)),
            pl.BlockSpec((o, c), lambda i: (0, 0)),
            pl.BlockSpec((1, o), lambda i: (0, 0)),
        ],
        out_specs=[
            pl.BlockSpec((blk, o), lambda i: (i, 0)),
            pl.BlockSpec((blk, o), lambda i: (i, 0)),
        ],
        out_shape=[
            jax.ShapeDtypeStruct((m, o), jnp.float32),
            jax.ShapeDtypeStruct((m, o), jnp.float32),
        ],
    )(xf, wa, wb, bias.reshape(1, o))


def _gather_sub(idx2d, xf):
    m, c = xf.shape
    per_w = m // _NW
    g = 16
    t = per_w // g
    nsl = (g * _K) // 80
    mesh = plsc.VectorSubcoreMesh(core_axis_name="c", subcore_axis_name="s",
                                  num_cores=_NC, num_subcores=_NS)

    @functools.partial(
        pl.kernel,
        out_type=jax.ShapeDtypeStruct((_K, m, c), jnp.float32),
        mesh=mesh,
        scratch_types=[
            pltpu.VMEM((2, nsl, 80), jnp.int32),
            pltpu.VMEM((2, g * _K, c), jnp.float32),
            pltpu.VMEM((2, g, c), jnp.float32),
            pltpu.VMEM((2, _K, g, c), jnp.float32),
            pltpu.SemaphoreType.DMA,
            pltpu.SemaphoreType.DMA,
            pltpu.SemaphoreType.DMA,
            pltpu.SemaphoreType.DMA,
            pltpu.SemaphoreType.DMA,
            pltpu.SemaphoreType.DMA,
        ],
    )
    def kern(idx_hbm, x_hbm, e_hbm, idx_v, rows_v, xi_v, o3_v,
             si0, si1, sg0, sg1, so0, so1):
        wid = lax.axis_index("c") * _NS + lax.axis_index("s")
        si = (si0, si1)
        sg = (sg0, sg1)
        so = (so0, so1)

        def idx_row(ch):
            return wid * (per_w * _K // 80) + ch * nsl

        def pt_base(ch):
            return wid * per_w + ch * g

        def fire_idx(ch, p):
            pltpu.async_copy(idx_hbm.at[pl.ds(idx_row(ch), nsl)],
                             idx_v.at[p], si[p])

        def wait_idx(p):
            pltpu.make_async_copy(idx_hbm.at[pl.ds(0, nsl)],
                                  idx_v.at[p], si[p]).wait()

        def fire_gathers(ch, p):
            for j in range(nsl):
                pltpu.async_copy(x_hbm.at[idx_v.at[p, j]],
                                 rows_v.at[p, pl.ds(j * 80, 80)], sg[p])
            pltpu.async_copy(x_hbm.at[pl.ds(pt_base(ch), g)], xi_v.at[p], sg[p])

        def wait_gathers(p):
            for j in range(nsl):
                pltpu.make_async_copy(x_hbm.at[idx_v.at[p, j]],
                                      rows_v.at[p, pl.ds(j * 80, 80)],
                                      sg[p]).wait()
            pltpu.make_async_copy(x_hbm.at[pl.ds(0, g)], xi_v.at[p],
                                  sg[p]).wait()

        def fire_outs(ch, p):
            for r in range(_K):
                pltpu.async_copy(o3_v.at[p, r],
                                 e_hbm.at[r, pl.ds(pt_base(ch), g)], so[p])

        def wait_outs(p):
            for r in range(_K):
                pltpu.make_async_copy(o3_v.at[p, r],
                                      e_hbm.at[r, pl.ds(0, g)], so[p]).wait()

        def compute(p):
            def cbody(gg, carry):
                for cc in range(c // 16):
                    sl = pl.ds(cc * 16, 16)
                    xiv = xi_v[p, gg, sl]
                    for r in range(_K):
                        o3_v[p, r, gg, sl] = rows_v[p, gg * _K + r, sl] - xiv
                return carry
            lax.fori_loop(0, g, cbody, 0)

        def chunk_step(ch, p, drain_outs):
            q = 1 - p
            if drain_outs:
                wait_outs(p)
            wait_gathers(p)
            nxt = jnp.minimum(ch + 1, t - 1)
            wait_idx(q)
            fire_gathers(nxt, q)
            fire_idx(jnp.minimum(ch + 2, t - 1), p)
            compute(p)
            fire_outs(ch, p)

        fire_idx(0, 0)
        wait_idx(0)
        fire_gathers(0, 0)
        fire_idx(1, 1)
        chunk_step(0, 0, False)
        chunk_step(1, 1, False)

        def body(u, carry):
            chunk_step(2 * u, 0, True)
            chunk_step(2 * u + 1, 1, True)
            return carry

        lax.fori_loop(1, t // 2, body, 0)

        wait_outs(0)
        wait_outs(1)
        wait_gathers(0)
        wait_idx(1)

    return kern(idx2d, xf)


def _edge_mlp_body(e_ref, xi_ref, w_ref, b_ref, o_ref):
    e3 = e_ref[...]
    xi = xi_ref[...]
    k, blk, c = e3.shape
    xib = jnp.broadcast_to(xi[None], (k, blk, c))
    edge = jnp.concatenate([xib, e3], axis=2).reshape(k * blk, 2 * c)
    h = lax.dot_general(edge, w_ref[...], (((1,), (1,)), ((), ())),
                        preferred_element_type=jnp.float32)
    h = h + b_ref[...]
    h = jnp.where(h >= 0, h, 0.2 * h)
    o_ref[...] = jnp.max(h.reshape(k, blk, -1), axis=0)


def _edge_mlp(e3, xf, w, bias, blk=128):
    m, c = xf.shape
    o = w.shape[0]
    return pl.pallas_call(
        _edge_mlp_body,
        grid=(m // blk,),
        in_specs=[
            pl.BlockSpec((_K, blk, c), lambda i: (0, i, 0)),
            pl.BlockSpec((blk, c), lambda i: (i, 0)),
            pl.BlockSpec((o, 2 * c), lambda i: (0, 0)),
            pl.BlockSpec((1, o), lambda i: (0, 0)),
        ],
        out_specs=pl.BlockSpec((blk, o), lambda i: (i, 0)),
        out_shape=jax.ShapeDtypeStruct((m, o), jnp.float32),
    )(e3, xf, w, bias.reshape(1, o))


def _gather_max(idx2d, q, p):
    m, c = q.shape
    per_w = m // _NW
    g = 16
    t = per_w // g
    nsl = (g * _K) // 80
    mesh = plsc.VectorSubcoreMesh(core_axis_name="c", subcore_axis_name="s",
                                  num_cores=_NC, num_subcores=_NS)

    @functools.partial(
        pl.kernel,
        out_type=jax.ShapeDtypeStruct((m, c), jnp.float32),
        mesh=mesh,
        scratch_types=[
            pltpu.VMEM((2, nsl, 80), jnp.int32),
            pltpu.VMEM((2, g * _K, c), jnp.float32),
            pltpu.VMEM((2, g, c), jnp.float32),
            pltpu.VMEM((2, g, c), jnp.float32),
            pltpu.SemaphoreType.DMA,
            pltpu.SemaphoreType.DMA,
            pltpu.SemaphoreType.DMA,
            pltpu.SemaphoreType.DMA,
            pltpu.SemaphoreType.DMA,
            pltpu.SemaphoreType.DMA,
        ],
    )
    def kern(idx_hbm, q_hbm, p_hbm, out_hbm, idx_v, rows_v, p_v, o_v,
             si0, si1, sg0, sg1, so0, so1):
        wid = lax.axis_index("c") * _NS + lax.axis_index("s")
        si = (si0, si1)
        sg = (sg0, sg1)
        so = (so0, so1)

        def idx_row(ch):
            return wid * (per_w * _K // 80) + ch * nsl

        def pt_base(ch):
            return wid * per_w + ch * g

        def fire_idx(ch, pp):
            pltpu.async_copy(idx_hbm.at[pl.ds(idx_row(ch), nsl)],
                             idx_v.at[pp], si[pp])

        def wait_idx(pp):
            pltpu.make_async_copy(idx_hbm.at[pl.ds(0, nsl)],
                                  idx_v.at[pp], si[pp]).wait()

        def fire_gathers(ch, pp):
            for j in range(nsl):
                pltpu.async_copy(q_hbm.at[idx_v.at[pp, j]],
                                 rows_v.at[pp, pl.ds(j * 80, 80)], sg[pp])
            pltpu.async_copy(p_hbm.at[pl.ds(pt_base(ch), g)], p_v.at[pp],
                             sg[pp])

        def wait_gathers(pp):
            for j in range(nsl):
                pltpu.make_async_copy(q_hbm.at[idx_v.at[pp, j]],
                                      rows_v.at[pp, pl.ds(j * 80, 80)],
                                      sg[pp]).wait()
            pltpu.make_async_copy(p_hbm.at[pl.ds(0, g)], p_v.at[pp],
                                  sg[pp]).wait()

        def fire_outs(ch, pp):
            pltpu.async_copy(o_v.at[pp], out_hbm.at[pl.ds(pt_base(ch), g)],
                             so[pp])

        def wait_outs(pp):
            pltpu.make_async_copy(o_v.at[pp], out_hbm.at[pl.ds(0, g)],
                                  so[pp]).wait()

        def compute(pp):
            def cbody(gg, carry):
                for cc in range(c // 16):
                    sl = pl.ds(cc * 16, 16)
                    acc = rows_v[pp, gg * _K, sl]
                    for r in range(1, _K):
                        acc = jnp.maximum(acc, rows_v[pp, gg * _K + r, sl])
                    h = p_v[pp, gg, sl] + acc
                    o_v[pp, gg, sl] = jnp.maximum(h, 0.2 * h)
                return carry
            lax.fori_loop(0, g, cbody, 0)

        def chunk_step(ch, pp, drain_outs):
            qq = 1 - pp
            if drain_outs:
                wait_outs(pp)
            wait_gathers(pp)
            nxt = jnp.minimum(ch + 1, t - 1)
            wait_idx(qq)
            fire_gathers(nxt, qq)
            fire_idx(jnp.minimum(ch + 2, t - 1), pp)
            compute(pp)
            fire_outs(ch, pp)

        fire_idx(0, 0)
        wait_idx(0)
        fire_gathers(0, 0)
        fire_idx(1, 1)
        chunk_step(0, 0, False)
        chunk_step(1, 1, False)

        def body(u, carry):
            chunk_step(2 * u, 0, True)
            chunk_step(2 * u + 1, 1, True)
            return carry

        lax.fori_loop(1, t // 2, body, 0)

        wait_outs(0)
        wait_outs(1)
        wait_gathers(0)
        wait_idx(1)

    return kern(idx2d, q, p)


def _pw_body(x0_ref, x1_ref, x2_ref, wa_ref, wb_ref, wc_ref, b_ref, o_ref):
    dn = (((1,), (1,)), ((), ()))
    acc = lax.dot_general(x0_ref[...], wa_ref[...], dn,
                          preferred_element_type=jnp.float32)
    acc += lax.dot_general(x1_ref[...], wb_ref[...], dn,
                           preferred_element_type=jnp.float32)
    acc += lax.dot_general(x2_ref[...], wc_ref[...], dn,
                           preferred_element_type=jnp.float32)
    acc += b_ref[...]
    o_ref[...] = jnp.maximum(acc, 0.2 * acc)


def _pointwise(x0f, x1f, x2f, w3, b3, blk=512):
    m, c0 = x0f.shape
    c1 = x1f.shape[1]
    c2 = x2f.shape[1]
    o = w3.shape[0]
    wa = w3[:, :c0]
    wb = w3[:, c0:c0 + c1]
    wc = w3[:, c0 + c1:]
    return pl.pallas_call(
        _pw_body,
        grid=(m // blk,),
        in_specs=[
            pl.BlockSpec((blk, c0), lambda i: (i, 0)),
            pl.BlockSpec((blk, c1), lambda i: (i, 0)),
            pl.BlockSpec((blk, c2), lambda i: (i, 0)),
            pl.BlockSpec((o, c0), lambda i: (0, 0)),
            pl.BlockSpec((o, c1), lambda i: (0, 0)),
            pl.BlockSpec((o, c2), lambda i: (0, 0)),
            pl.BlockSpec((1, o), lambda i: (0, 0)),
        ],
        out_specs=pl.BlockSpec((blk, o), lambda i: (i, 0)),
        out_shape=jax.ShapeDtypeStruct((m, o), jnp.float32),
    )(x0f, x1f, x2f, wa, wb, wc, b3.reshape(1, o))


def _edgeconv(xt, w, bias):
    b, n, c = xt.shape
    idx = _topk(xt)
    idx10 = idx[:, :, :_K].reshape(-1, 80)
    xf = xt.reshape(b * n, c)
    p, q = _proj(xf, w[:, :c], w[:, c:], bias)
    xo = _gather_max(idx10, q, p)
    return xo.reshape(b, n, -1)


def kernel(features, W1, b1, W2, b2, W3, b3):
    b, f, n = features.shape
    x0t = jnp.transpose(features, (0, 2, 1))
    x0f = x0t.reshape(b * n, f)
    idx1 = _topk(x0t)[:, :, :_K].reshape(-1, 80)
    e3 = _gather_sub(idx1, x0f)
    x1t = _edge_mlp(e3, x0f, W1, b1).reshape(b, n, f)
    x2t = _edgeconv(x1t, W2, b2)
    m = b * n
    out = _pointwise(x0t.reshape(m, -1), x1t.reshape(m, -1),
                     x2t.reshape(m, -1), W3, b3)
    return jnp.transpose(out.reshape(b, n, f), (0, 2, 1))


# speedup vs baseline: 15.3115x; 1.2384x over previous
import functools

import jax
import jax.numpy as jnp
from jax import lax
from jax.experimental import pallas as pl
from jax.experimental.pallas import tpu as pltpu
from jax.experimental.pallas import tpu_sc as plsc

_K = 10
_KPAD = 16
_NEG = -3.0e38

_NC = 2
_NS = 16
_NW = _NC * _NS


def _topk_body(xr_ref, xc_ref, idx_ref):
    xr = xr_ref[0]
    xc = xc_ref[0]
    blk, _ = xr.shape
    n = xc.shape[0]
    sq = jnp.sum(xc * xc, axis=1)
    s = 2.0 * lax.dot_general(xr, xc, (((1,), (1,)), ((), ())),
                              preferred_element_type=jnp.float32)
    s = s - sq[None, :]
    fio = lax.broadcasted_iota(jnp.int32, (blk, n), 1).astype(jnp.float32)
    cols = []
    for _ in range(_K):
        m = jnp.max(s, axis=1, keepdims=True)
        hit = s == m
        am = jnp.min(jnp.where(hit, fio, 3.0e38), axis=1, keepdims=True)
        cols.append(am)
        s = jnp.where(hit, _NEG, s)
    self_idx = (pl.program_id(1) * blk
                + lax.broadcasted_iota(jnp.int32, (blk, 1), 0))
    base = pl.program_id(0) * n
    topf = jnp.concatenate(cols, axis=1).astype(jnp.int32)
    pad = jnp.concatenate([self_idx] * (_KPAD - _K), axis=1)
    idx_ref[0] = jnp.concatenate([topf, pad], axis=1) + base


def _topk(xt, blk=256):
    b, n, c = xt.shape
    return pl.pallas_call(
        _topk_body,
        grid=(b, n // blk),
        in_specs=[
            pl.BlockSpec((1, blk, c), lambda bi, i: (bi, i, 0)),
            pl.BlockSpec((1, n, c), lambda bi, i: (bi, 0, 0)),
        ],
        out_specs=pl.BlockSpec((1, blk, _KPAD), lambda bi, i: (bi, i, 0)),
        out_shape=jax.ShapeDtypeStruct((b, n, _KPAD), jnp.int32),
    )(xt, xt)


def _proj_body(x_ref, wa_ref, wb_ref, b_ref, p_ref, q_ref):
    x = x_ref[...]
    wb = wb_ref[...]
    wd = wa_ref[...] - wb
    dn = (((1,), (1,)), ((), ()))
    p_ref[...] = (lax.dot_general(x, wd, dn, preferred_element_type=jnp.float32)
                  + b_ref[...])
    q_ref[...] = lax.dot_general(x, wb, dn, preferred_element_type=jnp.float32)


def _proj(xf, wa, wb, bias, blk=512):
    m, c = xf.shape
    o = wa.shape[0]
    return pl.pallas_call(
        _proj_body,
        grid=(m // blk,),
        in_specs=[
            pl.BlockSpec((blk, c), lambda i: (i, 0)),
            pl.BlockSpec((o, c), lambda i: (0, 0)),
            pl.BlockSpec((o, c), lambda i: (0, 0)),
            pl.BlockSpec((1, o), lambda i: (0, 0)),
        ],
        out_specs=[
            pl.BlockSpec((blk, o), lambda i: (i, 0)),
            pl.BlockSpec((blk, o), lambda i: (i, 0)),
        ],
        out_shape=[
            jax.ShapeDtypeStruct((m, o), jnp.float32),
            jax.ShapeDtypeStruct((m, o), jnp.float32),
        ],
    )(xf, wa, wb, bias.reshape(1, o))


def _gather_sub(idx2d, xf):
    m, c = xf.shape
    per_w = m // _NW
    g = 16
    t = per_w // g
    nsl = (g * _K) // 80
    mesh = plsc.VectorSubcoreMesh(core_axis_name="c", subcore_axis_name="s",
                                  num_cores=_NC, num_subcores=_NS)

    @functools.partial(
        pl.kernel,
        out_type=jax.ShapeDtypeStruct((_K, m, c), jnp.float32),
        mesh=mesh,
        scratch_types=[
            pltpu.VMEM((2, nsl, 80), jnp.int32),
            pltpu.VMEM((2, g * _K, c), jnp.float32),
            pltpu.VMEM((2, g, c), jnp.float32),
            pltpu.VMEM((2, _K, g, c), jnp.float32),
            pltpu.SemaphoreType.DMA,
            pltpu.SemaphoreType.DMA,
            pltpu.SemaphoreType.DMA,
            pltpu.SemaphoreType.DMA,
            pltpu.SemaphoreType.DMA,
            pltpu.SemaphoreType.DMA,
        ],
    )
    def kern(idx_hbm, x_hbm, e_hbm, idx_v, rows_v, xi_v, o3_v,
             si0, si1, sg0, sg1, so0, so1):
        wid = lax.axis_index("c") * _NS + lax.axis_index("s")
        si = (si0, si1)
        sg = (sg0, sg1)
        so = (so0, so1)

        def idx_row(ch):
            return wid * (per_w * _K // 80) + ch * nsl

        def pt_base(ch):
            return wid * per_w + ch * g

        def fire_idx(ch, p):
            pltpu.async_copy(idx_hbm.at[pl.ds(idx_row(ch), nsl)],
                             idx_v.at[p], si[p])

        def wait_idx(p):
            pltpu.make_async_copy(idx_hbm.at[pl.ds(0, nsl)],
                                  idx_v.at[p], si[p]).wait()

        def fire_gathers(ch, p):
            for j in range(nsl):
                pltpu.async_copy(x_hbm.at[idx_v.at[p, j]],
                                 rows_v.at[p, pl.ds(j * 80, 80)], sg[p])
            pltpu.async_copy(x_hbm.at[pl.ds(pt_base(ch), g)], xi_v.at[p], sg[p])

        def wait_gathers(p):
            for j in range(nsl):
                pltpu.make_async_copy(x_hbm.at[idx_v.at[p, j]],
                                      rows_v.at[p, pl.ds(j * 80, 80)],
                                      sg[p]).wait()
            pltpu.make_async_copy(x_hbm.at[pl.ds(0, g)], xi_v.at[p],
                                  sg[p]).wait()

        def fire_outs(ch, p):
            for r in range(_K):
                pltpu.async_copy(o3_v.at[p, r],
                                 e_hbm.at[r, pl.ds(pt_base(ch), g)], so[p])

        def wait_outs(p):
            for r in range(_K):
                pltpu.make_async_copy(o3_v.at[p, r],
                                      e_hbm.at[r, pl.ds(0, g)], so[p]).wait()

        def compute(p):
            def cbody(gg, carry):
                for cc in range(c // 16):
                    sl = pl.ds(cc * 16, 16)
                    xiv = xi_v[p, gg, sl]
                    for r in range(_K):
                        o3_v[p, r, gg, sl] = rows_v[p, gg * _K + r, sl] - xiv
                return carry
            lax.fori_loop(0, g, cbody, 0)

        def chunk_step(ch, p, drain_outs):
            q = 1 - p
            if drain_outs:
                wait_outs(p)
            wait_gathers(p)
            nxt = jnp.minimum(ch + 1, t - 1)
            wait_idx(q)
            fire_gathers(nxt, q)
            fire_idx(jnp.minimum(ch + 2, t - 1), p)
            compute(p)
            fire_outs(ch, p)

        fire_idx(0, 0)
        wait_idx(0)
        fire_gathers(0, 0)
        fire_idx(1, 1)
        chunk_step(0, 0, False)
        chunk_step(1, 1, False)

        def body(u, carry):
            chunk_step(2 * u, 0, True)
            chunk_step(2 * u + 1, 1, True)
            return carry

        lax.fori_loop(1, t // 2, body, 0)

        wait_outs(0)
        wait_outs(1)
        wait_gathers(0)
        wait_idx(1)

    return kern(idx2d, xf)


def _edge_mlp_body(e_ref, xi_ref, w_ref, b_ref, o_ref):
    e3 = e_ref[...]
    xi = xi_ref[...]
    k, blk, c = e3.shape
    xib = jnp.broadcast_to(xi[None], (k, blk, c))
    edge = jnp.concatenate([xib, e3], axis=2).reshape(k * blk, 2 * c)
    h = lax.dot_general(edge, w_ref[...], (((1,), (1,)), ((), ())),
                        preferred_element_type=jnp.float32)
    h = h + b_ref[...]
    h = jnp.where(h >= 0, h, 0.2 * h)
    o_ref[...] = jnp.max(h.reshape(k, blk, -1), axis=0)


def _edge_mlp(e3, xf, w, bias, blk=128):
    m, c = xf.shape
    o = w.shape[0]
    return pl.pallas_call(
        _edge_mlp_body,
        grid=(m // blk,),
        in_specs=[
            pl.BlockSpec((_K, blk, c), lambda i: (0, i, 0)),
            pl.BlockSpec((blk, c), lambda i: (i, 0)),
            pl.BlockSpec((o, 2 * c), lambda i: (0, 0)),
            pl.BlockSpec((1, o), lambda i: (0, 0)),
        ],
        out_specs=pl.BlockSpec((blk, o), lambda i: (i, 0)),
        out_shape=jax.ShapeDtypeStruct((m, o), jnp.float32),
    )(e3, xf, w, bias.reshape(1, o))


def _gather_max(idx2d, q, p):
    m, c = q.shape
    per_w = m // _NW
    g = 16
    t = per_w // g
    nsl = (g * _K) // 80
    mesh = plsc.VectorSubcoreMesh(core_axis_name="c", subcore_axis_name="s",
                                  num_cores=_NC, num_subcores=_NS)

    @functools.partial(
        pl.kernel,
        out_type=jax.ShapeDtypeStruct((m, c), jnp.float32),
        mesh=mesh,
        scratch_types=[
            pltpu.VMEM((2, nsl, 80), jnp.int32),
            pltpu.VMEM((2, g * _K, c), jnp.float32),
            pltpu.VMEM((2, g, c), jnp.float32),
            pltpu.VMEM((2, g, c), jnp.float32),
            pltpu.SemaphoreType.DMA,
            pltpu.SemaphoreType.DMA,
            pltpu.SemaphoreType.DMA,
            pltpu.SemaphoreType.DMA,
            pltpu.SemaphoreType.DMA,
            pltpu.SemaphoreType.DMA,
        ],
    )
    def kern(idx_hbm, q_hbm, p_hbm, out_hbm, idx_v, rows_v, p_v, o_v,
             si0, si1, sg0, sg1, so0, so1):
        wid = lax.axis_index("c") * _NS + lax.axis_index("s")
        si = (si0, si1)
        sg = (sg0, sg1)
        so = (so0, so1)

        def idx_row(ch):
            return wid * (per_w * _K // 80) + ch * nsl

        def pt_base(ch):
            return wid * per_w + ch * g

        def fire_idx(ch, pp):
            pltpu.async_copy(idx_hbm.at[pl.ds(idx_row(ch), nsl)],
                             idx_v.at[pp], si[pp])

        def wait_idx(pp):
            pltpu.make_async_copy(idx_hbm.at[pl.ds(0, nsl)],
                                  idx_v.at[pp], si[pp]).wait()

        def fire_gathers(ch, pp):
            for j in range(nsl):
                pltpu.async_copy(q_hbm.at[idx_v.at[pp, j]],
                                 rows_v.at[pp, pl.ds(j * 80, 80)], sg[pp])
            pltpu.async_copy(p_hbm.at[pl.ds(pt_base(ch), g)], p_v.at[pp],
                             sg[pp])

        def wait_gathers(pp):
            for j in range(nsl):
                pltpu.make_async_copy(q_hbm.at[idx_v.at[pp, j]],
                                      rows_v.at[pp, pl.ds(j * 80, 80)],
                                      sg[pp]).wait()
            pltpu.make_async_copy(p_hbm.at[pl.ds(0, g)], p_v.at[pp],
                                  sg[pp]).wait()

        def fire_outs(ch, pp):
            pltpu.async_copy(o_v.at[pp], out_hbm.at[pl.ds(pt_base(ch), g)],
                             so[pp])

        def wait_outs(pp):
            pltpu.make_async_copy(o_v.at[pp], out_hbm.at[pl.ds(0, g)],
                                  so[pp]).wait()

        def compute(pp):
            def cbody(gg, carry):
                for cc in range(c // 16):
                    sl = pl.ds(cc * 16, 16)
                    acc = rows_v[pp, gg * _K, sl]
                    for r in range(1, _K):
                        acc = jnp.maximum(acc, rows_v[pp, gg * _K + r, sl])
                    h = p_v[pp, gg, sl] + acc
                    o_v[pp, gg, sl] = jnp.maximum(h, 0.2 * h)
                return carry
            lax.fori_loop(0, g, cbody, 0)

        def chunk_step(ch, pp, drain_outs):
            qq = 1 - pp
            if drain_outs:
                wait_outs(pp)
            wait_gathers(pp)
            nxt = jnp.minimum(ch + 1, t - 1)
            wait_idx(qq)
            fire_gathers(nxt, qq)
            fire_idx(jnp.minimum(ch + 2, t - 1), pp)
            compute(pp)
            fire_outs(ch, pp)

        fire_idx(0, 0)
        wait_idx(0)
        fire_gathers(0, 0)
        fire_idx(1, 1)
        chunk_step(0, 0, False)
        chunk_step(1, 1, False)

        def body(u, carry):
            chunk_step(2 * u, 0, True)
            chunk_step(2 * u + 1, 1, True)
            return carry

        lax.fori_loop(1, t // 2, body, 0)

        wait_outs(0)
        wait_outs(1)
        wait_gathers(0)
        wait_idx(1)

    return kern(idx2d, q, p)


def _pw_body(x0_ref, x1_ref, x2_ref, wa_ref, wb_ref, wc_ref, b_ref, o_ref):
    dn = (((1,), (1,)), ((), ()))
    acc = lax.dot_general(x0_ref[...], wa_ref[...], dn,
                          preferred_element_type=jnp.float32)
    acc += lax.dot_general(x1_ref[...], wb_ref[...], dn,
                           preferred_element_type=jnp.float32)
    acc += lax.dot_general(x2_ref[...], wc_ref[...], dn,
                           preferred_element_type=jnp.float32)
    acc += b_ref[...]
    o_ref[...] = jnp.maximum(acc, 0.2 * acc)


def _pointwise(x0f, x1f, x2f, w3, b3, blk=512):
    m, c0 = x0f.shape
    c1 = x1f.shape[1]
    c2 = x2f.shape[1]
    o = w3.shape[0]
    wa = w3[:, :c0]
    wb = w3[:, c0:c0 + c1]
    wc = w3[:, c0 + c1:]
    return pl.pallas_call(
        _pw_body,
        grid=(m // blk,),
        in_specs=[
            pl.BlockSpec((blk, c0), lambda i: (i, 0)),
            pl.BlockSpec((blk, c1), lambda i: (i, 0)),
            pl.BlockSpec((blk, c2), lambda i: (i, 0)),
            pl.BlockSpec((o, c0), lambda i: (0, 0)),
            pl.BlockSpec((o, c1), lambda i: (0, 0)),
            pl.BlockSpec((o, c2), lambda i: (0, 0)),
            pl.BlockSpec((1, o), lambda i: (0, 0)),
        ],
        out_specs=pl.BlockSpec((blk, o), lambda i: (i, 0)),
        out_shape=jax.ShapeDtypeStruct((m, o), jnp.float32),
    )(x0f, x1f, x2f, wa, wb, wc, b3.reshape(1, o))


def _edgeconv(xt, w, bias):
    b, n, c = xt.shape
    idx = _topk(xt)
    idx10 = idx[:, :, :_K].reshape(-1, 80)
    xf = xt.reshape(b * n, c)
    p, q = _proj(xf, w[:, :c], w[:, c:], bias)
    xo = _gather_max(idx10, q, p)
    return xo.reshape(b, n, -1)


def kernel(features, W1, b1, W2, b2, W3, b3):
    b, f, n = features.shape
    x0t = jnp.transpose(features, (0, 2, 1))
    x0f = x0t.reshape(b * n, f)
    idx1 = _topk(x0t)[:, :, :_K].reshape(-1, 80)
    e3 = _gather_sub(idx1, x0f)
    x1t = _edge_mlp(e3, x0f, W1, b1).reshape(b, n, f)
    x2t = _edgeconv(x1t, W2, b2)
    m = b * n
    out = _pointwise(x0t.reshape(m, -1), x1t.reshape(m, -1),
                     x2t.reshape(m, -1), W3, b3)
    return jnp.transpose(out.reshape(b, n, f), (0, 2, 1))


# speedup vs baseline: 17.5101x; 1.1436x over previous
import functools

import jax
import jax.numpy as jnp
from jax import lax
from jax.experimental import pallas as pl
from jax.experimental.pallas import tpu as pltpu
from jax.experimental.pallas import tpu_sc as plsc

_K = 10
_KPAD = 16
_NEG = -3.0e38

_NC = 2
_NS = 16
_NW = _NC * _NS


def _topk_body(xr_ref, xc_ref, idx_ref):
    xr = xr_ref[0]
    xc = xc_ref[0]
    blk, _ = xr.shape
    n = xc.shape[0]
    sq = jnp.sum(xc * xc, axis=1)
    s = 2.0 * lax.dot_general(xr, xc, (((1,), (1,)), ((), ())),
                              preferred_element_type=jnp.float32)
    s = s - sq[None, :]
    fio = lax.broadcasted_iota(jnp.int32, (blk, n), 1).astype(jnp.float32)
    cols = []
    for _ in range(_K):
        m = jnp.max(s, axis=1, keepdims=True)
        hit = s == m
        am = jnp.min(jnp.where(hit, fio, 3.0e38), axis=1, keepdims=True)
        cols.append(am)
        s = jnp.where(hit, _NEG, s)
    self_idx = (pl.program_id(1) * blk
                + lax.broadcasted_iota(jnp.int32, (blk, 1), 0))
    base = pl.program_id(0) * n
    topf = jnp.concatenate(cols, axis=1).astype(jnp.int32)
    pad = jnp.concatenate([self_idx] * (_KPAD - _K), axis=1)
    idx_ref[0] = jnp.concatenate([topf, pad], axis=1) + base


def _topk(xt, blk=256):
    b, n, c = xt.shape
    return pl.pallas_call(
        _topk_body,
        grid=(b, n // blk),
        in_specs=[
            pl.BlockSpec((1, blk, c), lambda bi, i: (bi, i, 0)),
            pl.BlockSpec((1, n, c), lambda bi, i: (bi, 0, 0)),
        ],
        out_specs=pl.BlockSpec((1, blk, _KPAD), lambda bi, i: (bi, i, 0)),
        out_shape=jax.ShapeDtypeStruct((b, n, _KPAD), jnp.int32),
    )(xt, xt)


def _proj_body(x_ref, wa_ref, wb_ref, b_ref, p_ref, q_ref):
    x = x_ref[...]
    wb = wb_ref[...]
    wd = wa_ref[...] - wb
    dn = (((1,), (1,)), ((), ()))
    p_ref[...] = (lax.dot_general(x, wd, dn, preferred_element_type=jnp.float32)
                  + b_ref[...])
    q_ref[...] = lax.dot_general(x, wb, dn, preferred_element_type=jnp.float32)


def _proj(xf, wa, wb, bias, blk=512):
    m, c = xf.shape
    o = wa.shape[0]
    return pl.pallas_call(
        _proj_body,
        grid=(m // blk,),
        in_specs=[
            pl.BlockSpec((blk, c), lambda i: (i, 0)),
            pl.BlockSpec((o, c), lambda i: (0, 0)),
            pl.BlockSpec((o, c), lambda i: (0, 0)),
            pl.BlockSpec((1, o), lambda i: (0, 0)),
        ],
        out_specs=[
            pl.BlockSpec((blk, o), lambda i: (i, 0)),
            pl.BlockSpec((blk, o), lambda i: (i, 0)),
        ],
        out_shape=[
            jax.ShapeDtypeStruct((m, o), jnp.float32),
            jax.ShapeDtypeStruct((m, o), jnp.float32),
        ],
    )(xf, wa, wb, bias.reshape(1, o))


def _gather_sub(idx2d, xf):
    m, c = xf.shape
    per_w = m // _NW
    g = 16
    t = per_w // g
    nsl = (g * _K) // 80
    mesh = plsc.VectorSubcoreMesh(core_axis_name="c", subcore_axis_name="s",
                                  num_cores=_NC, num_subcores=_NS)

    @functools.partial(
        pl.kernel,
        out_type=jax.ShapeDtypeStruct((_K, m, c), jnp.float32),
        mesh=mesh,
        scratch_types=[
            pltpu.VMEM((2, nsl, 80), jnp.int32),
            pltpu.VMEM((2, g * _K, c), jnp.float32),
            pltpu.VMEM((2, g, c), jnp.float32),
            pltpu.VMEM((2, _K, g, c), jnp.float32),
            pltpu.SemaphoreType.DMA,
            pltpu.SemaphoreType.DMA,
            pltpu.SemaphoreType.DMA,
            pltpu.SemaphoreType.DMA,
            pltpu.SemaphoreType.DMA,
            pltpu.SemaphoreType.DMA,
        ],
    )
    def kern(idx_hbm, x_hbm, e_hbm, idx_v, rows_v, xi_v, o3_v,
             si0, si1, sg0, sg1, so0, so1):
        wid = lax.axis_index("c") * _NS + lax.axis_index("s")
        si = (si0, si1)
        sg = (sg0, sg1)
        so = (so0, so1)

        def idx_row(ch):
            return wid * (per_w * _K // 80) + ch * nsl

        def pt_base(ch):
            return wid * per_w + ch * g

        def fire_idx(ch, p):
            pltpu.async_copy(idx_hbm.at[pl.ds(idx_row(ch), nsl)],
                             idx_v.at[p], si[p])

        def wait_idx(p):
            pltpu.make_async_copy(idx_hbm.at[pl.ds(0, nsl)],
                                  idx_v.at[p], si[p]).wait()

        def fire_gathers(ch, p):
            for j in range(nsl):
                pltpu.async_copy(x_hbm.at[idx_v.at[p, j]],
                                 rows_v.at[p, pl.ds(j * 80, 80)], sg[p])
            pltpu.async_copy(x_hbm.at[pl.ds(pt_base(ch), g)], xi_v.at[p], sg[p])

        def wait_gathers(p):
            for j in range(nsl):
                pltpu.make_async_copy(x_hbm.at[idx_v.at[p, j]],
                                      rows_v.at[p, pl.ds(j * 80, 80)],
                                      sg[p]).wait()
            pltpu.make_async_copy(x_hbm.at[pl.ds(0, g)], xi_v.at[p],
                                  sg[p]).wait()

        def fire_outs(ch, p):
            for r in range(_K):
                pltpu.async_copy(o3_v.at[p, r],
                                 e_hbm.at[r, pl.ds(pt_base(ch), g)], so[p])

        def wait_outs(p):
            for r in range(_K):
                pltpu.make_async_copy(o3_v.at[p, r],
                                      e_hbm.at[r, pl.ds(0, g)], so[p]).wait()

        def compute(p):
            def cbody(gg, carry):
                for cc in range(c // 16):
                    sl = pl.ds(cc * 16, 16)
                    xiv = xi_v[p, gg, sl]
                    for r in range(_K):
                        o3_v[p, r, gg, sl] = rows_v[p, gg * _K + r, sl] - xiv
                return carry
            lax.fori_loop(0, g, cbody, 0)

        def chunk_step(ch, p, drain_outs):
            q = 1 - p
            if drain_outs:
                wait_outs(p)
            wait_gathers(p)
            nxt = jnp.minimum(ch + 1, t - 1)
            wait_idx(q)
            fire_gathers(nxt, q)
            fire_idx(jnp.minimum(ch + 2, t - 1), p)
            compute(p)
            fire_outs(ch, p)

        fire_idx(0, 0)
        wait_idx(0)
        fire_gathers(0, 0)
        fire_idx(1, 1)
        chunk_step(0, 0, False)
        chunk_step(1, 1, False)

        def body(u, carry):
            chunk_step(2 * u, 0, True)
            chunk_step(2 * u + 1, 1, True)
            return carry

        lax.fori_loop(1, t // 2, body, 0)

        wait_outs(0)
        wait_outs(1)
        wait_gathers(0)
        wait_idx(1)

    return kern(idx2d, xf)


def _edge_mlp_body(e_ref, xi_ref, w_ref, b_ref, o_ref):
    e3 = e_ref[...]
    xi = xi_ref[...]
    k, blk, c = e3.shape
    xib = jnp.broadcast_to(xi[None], (k, blk, c))
    edge = jnp.concatenate([xib, e3], axis=2).reshape(k * blk, 2 * c)
    h = lax.dot_general(edge, w_ref[...], (((1,), (1,)), ((), ())),
                        preferred_element_type=jnp.float32)
    h = h + b_ref[...]
    h = jnp.where(h >= 0, h, 0.2 * h)
    o_ref[...] = jnp.max(h.reshape(k, blk, -1), axis=0)


def _edge_mlp(e3, xf, w, bias, blk=128):
    m, c = xf.shape
    o = w.shape[0]
    return pl.pallas_call(
        _edge_mlp_body,
        grid=(m // blk,),
        in_specs=[
            pl.BlockSpec((_K, blk, c), lambda i: (0, i, 0)),
            pl.BlockSpec((blk, c), lambda i: (i, 0)),
            pl.BlockSpec((o, 2 * c), lambda i: (0, 0)),
            pl.BlockSpec((1, o), lambda i: (0, 0)),
        ],
        out_specs=pl.BlockSpec((blk, o), lambda i: (i, 0)),
        out_shape=jax.ShapeDtypeStruct((m, o), jnp.float32),
    )(e3, xf, w, bias.reshape(1, o))


def _gather_max(idx2d, q, p):
    m, c = q.shape
    per_w = m // _NW
    g = 16
    t = per_w // g
    nsl = (g * _K) // 80
    mesh = plsc.VectorSubcoreMesh(core_axis_name="c", subcore_axis_name="s",
                                  num_cores=_NC, num_subcores=_NS)

    @functools.partial(
        pl.kernel,
        out_type=jax.ShapeDtypeStruct((m, c), jnp.float32),
        mesh=mesh,
        scratch_types=[
            pltpu.VMEM((2, nsl, 80), jnp.int32),
            pltpu.VMEM((2, g * _K, c), jnp.float32),
            pltpu.VMEM((2, g, c), jnp.float32),
            pltpu.VMEM((2, g, c), jnp.float32),
            pltpu.SemaphoreType.DMA,
            pltpu.SemaphoreType.DMA,
            pltpu.SemaphoreType.DMA,
            pltpu.SemaphoreType.DMA,
            pltpu.SemaphoreType.DMA,
            pltpu.SemaphoreType.DMA,
        ],
    )
    def kern(idx_hbm, q_hbm, p_hbm, out_hbm, idx_v, rows_v, p_v, o_v,
             si0, si1, sg0, sg1, so0, so1):
        wid = lax.axis_index("c") * _NS + lax.axis_index("s")
        si = (si0, si1)
        sg = (sg0, sg1)
        so = (so0, so1)

        def idx_row(ch):
            return wid * (per_w * _K // 80) + ch * nsl

        def pt_base(ch):
            return wid * per_w + ch * g

        def fire_idx(ch, pp):
            pltpu.async_copy(idx_hbm.at[pl.ds(idx_row(ch), nsl)],
                             idx_v.at[pp], si[pp])

        def wait_idx(pp):
            pltpu.make_async_copy(idx_hbm.at[pl.ds(0, nsl)],
                                  idx_v.at[pp], si[pp]).wait()

        def fire_gathers(ch, pp):
            for j in range(nsl):
                pltpu.async_copy(q_hbm.at[idx_v.at[pp, j]],
                                 rows_v.at[pp, pl.ds(j * 80, 80)], sg[pp])
            pltpu.async_copy(p_hbm.at[pl.ds(pt_base(ch), g)], p_v.at[pp],
                             sg[pp])

        def wait_gathers(pp):
            for j in range(nsl):
                pltpu.make_async_copy(q_hbm.at[idx_v.at[pp, j]],
                                      rows_v.at[pp, pl.ds(j * 80, 80)],
                                      sg[pp]).wait()
            pltpu.make_async_copy(p_hbm.at[pl.ds(0, g)], p_v.at[pp],
                                  sg[pp]).wait()

        def fire_outs(ch, pp):
            pltpu.async_copy(o_v.at[pp], out_hbm.at[pl.ds(pt_base(ch), g)],
                             so[pp])

        def wait_outs(pp):
            pltpu.make_async_copy(o_v.at[pp], out_hbm.at[pl.ds(0, g)],
                                  so[pp]).wait()

        def compute(pp):
            def cbody(gg, carry):
                for cc in range(c // 16):
                    sl = pl.ds(cc * 16, 16)
                    acc = rows_v[pp, gg * _K, sl]
                    for r in range(1, _K):
                        acc = jnp.maximum(acc, rows_v[pp, gg * _K + r, sl])
                    h = p_v[pp, gg, sl] + acc
                    o_v[pp, gg, sl] = jnp.maximum(h, 0.2 * h)
                return carry
            lax.fori_loop(0, g, cbody, 0)

        def chunk_step(ch, pp, drain_outs):
            qq = 1 - pp
            if drain_outs:
                wait_outs(pp)
            wait_gathers(pp)
            nxt = jnp.minimum(ch + 1, t - 1)
            wait_idx(qq)
            fire_gathers(nxt, qq)
            fire_idx(jnp.minimum(ch + 2, t - 1), pp)
            compute(pp)
            fire_outs(ch, pp)

        fire_idx(0, 0)
        wait_idx(0)
        fire_gathers(0, 0)
        fire_idx(1, 1)
        chunk_step(0, 0, False)
        chunk_step(1, 1, False)

        def body(u, carry):
            chunk_step(2 * u, 0, True)
            chunk_step(2 * u + 1, 1, True)
            return carry

        lax.fori_loop(1, t // 2, body, 0)

        wait_outs(0)
        wait_outs(1)
        wait_gathers(0)
        wait_idx(1)

    return kern(idx2d, q, p)


def _pw_body(x0_ref, x1_ref, x2_ref, wa_ref, wb_ref, wc_ref, b_ref, o_ref):
    dn = (((1,), (1,)), ((), ()))
    acc = lax.dot_general(x0_ref[...], wa_ref[...], dn,
                          preferred_element_type=jnp.float32)
    acc += lax.dot_general(x1_ref[...], wb_ref[...], dn,
                           preferred_element_type=jnp.float32)
    acc += lax.dot_general(x2_ref[...], wc_ref[...], dn,
                           preferred_element_type=jnp.float32)
    acc += b_ref[...]
    o_ref[...] = jnp.maximum(acc, 0.2 * acc)


def _pointwise(x0f, x1f, x2f, w3, b3, blk=512):
    m, c0 = x0f.shape
    c1 = x1f.shape[1]
    c2 = x2f.shape[1]
    o = w3.shape[0]
    wa = w3[:, :c0]
    wb = w3[:, c0:c0 + c1]
    wc = w3[:, c0 + c1:]
    return pl.pallas_call(
        _pw_body,
        grid=(m // blk,),
        in_specs=[
            pl.BlockSpec((blk, c0), lambda i: (i, 0)),
            pl.BlockSpec((blk, c1), lambda i: (i, 0)),
            pl.BlockSpec((blk, c2), lambda i: (i, 0)),
            pl.BlockSpec((o, c0), lambda i: (0, 0)),
            pl.BlockSpec((o, c1), lambda i: (0, 0)),
            pl.BlockSpec((o, c2), lambda i: (0, 0)),
            pl.BlockSpec((1, o), lambda i: (0, 0)),
        ],
        out_specs=pl.BlockSpec((blk, o), lambda i: (i, 0)),
        out_shape=jax.ShapeDtypeStruct((m, o), jnp.float32),
    )(x0f, x1f, x2f, wa, wb, wc, b3.reshape(1, o))


def _edgeconv(xt, w, bias):
    b, n, c = xt.shape
    idx = _topk(xt)
    idx10 = idx[:, :, :_K].reshape(-1, 80)
    xf = xt.reshape(b * n, c)
    p, q = _proj(xf, w[:, :c], w[:, c:], bias)
    xo = _gather_max(idx10, q, p)
    return xo.reshape(b, n, -1)


def kernel(features, W1, b1, W2, b2, W3, b3):
    b, f, n = features.shape
    x0t_all = jnp.transpose(features, (0, 2, 1))
    outs = []
    for bi in range(b):
        x0t = lax.slice_in_dim(x0t_all, bi, bi + 1, axis=0)
        x0f = x0t.reshape(n, f)
        idx1 = _topk(x0t)[:, :, :_K].reshape(-1, 80)
        e3 = _gather_sub(idx1, x0f)
        x1f = _edge_mlp(e3, x0f, W1, b1)
        x2f = _edgeconv(x1f.reshape(1, n, f), W2, b2).reshape(n, 2 * f)
        outs.append(_pointwise(x0f, x1f, x2f, W3, b3))
    out = jnp.stack(outs)
    return jnp.transpose(out, (0, 2, 1))


# speedup vs baseline: 18.3558x; 1.0483x over previous
import functools

import jax
import jax.numpy as jnp
from jax import lax
from jax.experimental import pallas as pl
from jax.experimental.pallas import tpu as pltpu
from jax.experimental.pallas import tpu_sc as plsc

_K = 10
_KPAD = 16
_NEG = -3.0e38

_NC = 2
_NS = 16
_NW = _NC * _NS


def _topk_body(xr_ref, xc_ref, idx_ref):
    xr = xr_ref[0]
    xc = xc_ref[0]
    blk, _ = xr.shape
    n = xc.shape[0]
    sq = jnp.sum(xc * xc, axis=1)
    s = 2.0 * lax.dot_general(xr, xc, (((1,), (1,)), ((), ())),
                              preferred_element_type=jnp.float32)
    s = s - sq[None, :]
    fio = lax.broadcasted_iota(jnp.int32, (blk, n), 1).astype(jnp.float32)
    cols = []
    for _ in range(_K):
        m = jnp.max(s, axis=1, keepdims=True)
        hit = s == m
        am = jnp.min(jnp.where(hit, fio, 3.0e38), axis=1, keepdims=True)
        cols.append(am)
        s = jnp.where(hit, _NEG, s)
    self_idx = (pl.program_id(1) * blk
                + lax.broadcasted_iota(jnp.int32, (blk, 1), 0))
    base = pl.program_id(0) * n
    topf = jnp.concatenate(cols, axis=1).astype(jnp.int32)
    pad = jnp.concatenate([self_idx] * (_KPAD - _K), axis=1)
    idx_ref[0] = jnp.concatenate([topf, pad], axis=1) + base


def _topk(xt, blk=256):
    b, n, c = xt.shape
    return pl.pallas_call(
        _topk_body,
        grid=(b, n // blk),
        in_specs=[
            pl.BlockSpec((1, blk, c), lambda bi, i: (bi, i, 0)),
            pl.BlockSpec((1, n, c), lambda bi, i: (bi, 0, 0)),
        ],
        out_specs=pl.BlockSpec((1, blk, _KPAD), lambda bi, i: (bi, i, 0)),
        out_shape=jax.ShapeDtypeStruct((b, n, _KPAD), jnp.int32),
    )(xt, xt)


def _proj_body(x_ref, wa_ref, wb_ref, b_ref, p_ref, q_ref):
    x = x_ref[...]
    wb = wb_ref[...]
    wd = wa_ref[...] - wb
    dn = (((1,), (1,)), ((), ()))
    p_ref[...] = (lax.dot_general(x, wd, dn, preferred_element_type=jnp.float32)
                  + b_ref[...])
    q_ref[...] = lax.dot_general(x, wb, dn, preferred_element_type=jnp.float32)


def _proj(xf, wa, wb, bias, blk=512):
    m, c = xf.shape
    o = wa.shape[0]
    return pl.pallas_call(
        _proj_body,
        grid=(m // blk,),
        in_specs=[
            pl.BlockSpec((blk, c), lambda i: (i, 0)),
            pl.BlockSpec((o, c), lambda i: (0, 0)),
            pl.BlockSpec((o, c), lambda i: (0, 0)),
            pl.BlockSpec((1, o), lambda i: (0, 0)),
        ],
        out_specs=[
            pl.BlockSpec((blk, o), lambda i: (i, 0)),
            pl.BlockSpec((blk, o), lambda i: (i, 0)),
        ],
        out_shape=[
            jax.ShapeDtypeStruct((m, o), jnp.float32),
            jax.ShapeDtypeStruct((m, o), jnp.float32),
        ],
    )(xf, wa, wb, bias.reshape(1, o))


def _gather_sub(idx2d, xf):
    m, c = xf.shape
    per_w = m // _NW
    g = 16
    t = per_w // g
    nsl = (g * _K) // 80
    mesh = plsc.VectorSubcoreMesh(core_axis_name="c", subcore_axis_name="s",
                                  num_cores=_NC, num_subcores=_NS)

    @functools.partial(
        pl.kernel,
        out_type=jax.ShapeDtypeStruct((_K, m, c), jnp.float32),
        mesh=mesh,
        scratch_types=[
            pltpu.VMEM((2, nsl, 80), jnp.int32),
            pltpu.VMEM((2, g * _K, c), jnp.float32),
            pltpu.VMEM((2, g, c), jnp.float32),
            pltpu.VMEM((2, _K, g, c), jnp.float32),
            pltpu.SemaphoreType.DMA,
            pltpu.SemaphoreType.DMA,
            pltpu.SemaphoreType.DMA,
            pltpu.SemaphoreType.DMA,
            pltpu.SemaphoreType.DMA,
            pltpu.SemaphoreType.DMA,
        ],
    )
    def kern(idx_hbm, x_hbm, e_hbm, idx_v, rows_v, xi_v, o3_v,
             si0, si1, sg0, sg1, so0, so1):
        wid = lax.axis_index("c") * _NS + lax.axis_index("s")
        si = (si0, si1)
        sg = (sg0, sg1)
        so = (so0, so1)

        def idx_row(ch):
            return wid * (per_w * _K // 80) + ch * nsl

        def pt_base(ch):
            return wid * per_w + ch * g

        def fire_idx(ch, p):
            pltpu.async_copy(idx_hbm.at[pl.ds(idx_row(ch), nsl)],
                             idx_v.at[p], si[p])

        def wait_idx(p):
            pltpu.make_async_copy(idx_hbm.at[pl.ds(0, nsl)],
                                  idx_v.at[p], si[p]).wait()

        def fire_gathers(ch, p):
            for j in range(nsl):
                pltpu.async_copy(x_hbm.at[idx_v.at[p, j]],
                                 rows_v.at[p, pl.ds(j * 80, 80)], sg[p])
            pltpu.async_copy(x_hbm.at[pl.ds(pt_base(ch), g)], xi_v.at[p], sg[p])

        def wait_gathers(p):
            for j in range(nsl):
                pltpu.make_async_copy(x_hbm.at[idx_v.at[p, j]],
                                      rows_v.at[p, pl.ds(j * 80, 80)],
                                      sg[p]).wait()
            pltpu.make_async_copy(x_hbm.at[pl.ds(0, g)], xi_v.at[p],
                                  sg[p]).wait()

        def fire_outs(ch, p):
            for r in range(_K):
                pltpu.async_copy(o3_v.at[p, r],
                                 e_hbm.at[r, pl.ds(pt_base(ch), g)], so[p])

        def wait_outs(p):
            for r in range(_K):
                pltpu.make_async_copy(o3_v.at[p, r],
                                      e_hbm.at[r, pl.ds(0, g)], so[p]).wait()

        def compute(p):
            def cbody(gg, carry):
                for cc in range(c // 16):
                    sl = pl.ds(cc * 16, 16)
                    xiv = xi_v[p, gg, sl]
                    for r in range(_K):
                        o3_v[p, r, gg, sl] = rows_v[p, gg * _K + r, sl] - xiv
                return carry
            lax.fori_loop(0, g, cbody, 0)

        def chunk_step(ch, p, drain_outs):
            q = 1 - p
            if drain_outs:
                wait_outs(p)
            wait_gathers(p)
            nxt = jnp.minimum(ch + 1, t - 1)
            wait_idx(q)
            fire_gathers(nxt, q)
            fire_idx(jnp.minimum(ch + 2, t - 1), p)
            compute(p)
            fire_outs(ch, p)

        fire_idx(0, 0)
        wait_idx(0)
        fire_gathers(0, 0)
        fire_idx(1, 1)
        chunk_step(0, 0, False)
        chunk_step(1, 1, False)

        def body(u, carry):
            chunk_step(2 * u, 0, True)
            chunk_step(2 * u + 1, 1, True)
            return carry

        lax.fori_loop(1, t // 2, body, 0)

        wait_outs(0)
        wait_outs(1)
        wait_gathers(0)
        wait_idx(1)

    return kern(idx2d, xf)


def _edge_mlp_body(e_ref, xi_ref, w_ref, b_ref, o_ref):
    e3 = e_ref[...]
    xi = xi_ref[...]
    k, blk, c = e3.shape
    xib = jnp.broadcast_to(xi[None], (k, blk, c))
    edge = jnp.concatenate([xib, e3], axis=2).reshape(k * blk, 2 * c)
    h = lax.dot_general(edge, w_ref[...], (((1,), (1,)), ((), ())),
                        preferred_element_type=jnp.float32)
    h = h + b_ref[...]
    h = jnp.where(h >= 0, h, 0.2 * h)
    o_ref[...] = jnp.max(h.reshape(k, blk, -1), axis=0)


def _edge_mlp(e3, xf, w, bias, blk=128):
    m, c = xf.shape
    o = w.shape[0]
    return pl.pallas_call(
        _edge_mlp_body,
        grid=(m // blk,),
        in_specs=[
            pl.BlockSpec((_K, blk, c), lambda i: (0, i, 0)),
            pl.BlockSpec((blk, c), lambda i: (i, 0)),
            pl.BlockSpec((o, 2 * c), lambda i: (0, 0)),
            pl.BlockSpec((1, o), lambda i: (0, 0)),
        ],
        out_specs=pl.BlockSpec((blk, o), lambda i: (i, 0)),
        out_shape=jax.ShapeDtypeStruct((m, o), jnp.float32),
    )(e3, xf, w, bias.reshape(1, o))


def _gather_max(idx2d, q, p):
    m, c = q.shape
    per_w = m // _NW
    g = 16
    t = per_w // g
    nsl = (g * _K) // 80
    mesh = plsc.VectorSubcoreMesh(core_axis_name="c", subcore_axis_name="s",
                                  num_cores=_NC, num_subcores=_NS)

    @functools.partial(
        pl.kernel,
        out_type=jax.ShapeDtypeStruct((m, c), jnp.float32),
        mesh=mesh,
        scratch_types=[
            pltpu.VMEM((2, nsl, 80), jnp.int32),
            pltpu.VMEM((2, g * _K, c), jnp.float32),
            pltpu.VMEM((2, g, c), jnp.float32),
            pltpu.VMEM((2, g, c), jnp.float32),
            pltpu.SemaphoreType.DMA,
            pltpu.SemaphoreType.DMA,
            pltpu.SemaphoreType.DMA,
            pltpu.SemaphoreType.DMA,
            pltpu.SemaphoreType.DMA,
            pltpu.SemaphoreType.DMA,
        ],
    )
    def kern(idx_hbm, q_hbm, p_hbm, out_hbm, idx_v, rows_v, p_v, o_v,
             si0, si1, sg0, sg1, so0, so1):
        wid = lax.axis_index("c") * _NS + lax.axis_index("s")
        si = (si0, si1)
        sg = (sg0, sg1)
        so = (so0, so1)

        def idx_row(ch):
            return wid * (per_w * _K // 80) + ch * nsl

        def pt_base(ch):
            return wid * per_w + ch * g

        def fire_idx(ch, pp):
            pltpu.async_copy(idx_hbm.at[pl.ds(idx_row(ch), nsl)],
                             idx_v.at[pp], si[pp])

        def wait_idx(pp):
            pltpu.make_async_copy(idx_hbm.at[pl.ds(0, nsl)],
                                  idx_v.at[pp], si[pp]).wait()

        def fire_gathers(ch, pp):
            for j in range(nsl):
                pltpu.async_copy(q_hbm.at[idx_v.at[pp, j]],
                                 rows_v.at[pp, pl.ds(j * 80, 80)], sg[pp])
            pltpu.async_copy(p_hbm.at[pl.ds(pt_base(ch), g)], p_v.at[pp],
                             sg[pp])

        def wait_gathers(pp):
            for j in range(nsl):
                pltpu.make_async_copy(q_hbm.at[idx_v.at[pp, j]],
                                      rows_v.at[pp, pl.ds(j * 80, 80)],
                                      sg[pp]).wait()
            pltpu.make_async_copy(p_hbm.at[pl.ds(0, g)], p_v.at[pp],
                                  sg[pp]).wait()

        def fire_outs(ch, pp):
            pltpu.async_copy(o_v.at[pp], out_hbm.at[pl.ds(pt_base(ch), g)],
                             so[pp])

        def wait_outs(pp):
            pltpu.make_async_copy(o_v.at[pp], out_hbm.at[pl.ds(0, g)],
                                  so[pp]).wait()

        def compute(pp):
            def cbody(gg, carry):
                for cc in range(c // 16):
                    sl = pl.ds(cc * 16, 16)
                    acc = rows_v[pp, gg * _K, sl]
                    for r in range(1, _K):
                        acc = jnp.maximum(acc, rows_v[pp, gg * _K + r, sl])
                    h = p_v[pp, gg, sl] + acc
                    o_v[pp, gg, sl] = jnp.maximum(h, 0.2 * h)
                return carry
            lax.fori_loop(0, g, cbody, 0)

        def chunk_step(ch, pp, drain_outs):
            qq = 1 - pp
            if drain_outs:
                wait_outs(pp)
            wait_gathers(pp)
            nxt = jnp.minimum(ch + 1, t - 1)
            wait_idx(qq)
            fire_gathers(nxt, qq)
            fire_idx(jnp.minimum(ch + 2, t - 1), pp)
            compute(pp)
            fire_outs(ch, pp)

        fire_idx(0, 0)
        wait_idx(0)
        fire_gathers(0, 0)
        fire_idx(1, 1)
        chunk_step(0, 0, False)
        chunk_step(1, 1, False)

        def body(u, carry):
            chunk_step(2 * u, 0, True)
            chunk_step(2 * u + 1, 1, True)
            return carry

        lax.fori_loop(1, t // 2, body, 0)

        wait_outs(0)
        wait_outs(1)
        wait_gathers(0)
        wait_idx(1)

    return kern(idx2d, q, p)


def _pw_body(x0_ref, x1_ref, x2_ref, wa_ref, wb_ref, wc_ref, b_ref, o_ref):
    dn = (((1,), (1,)), ((), ()))
    acc = lax.dot_general(x0_ref[...], wa_ref[...], dn,
                          preferred_element_type=jnp.float32)
    acc += lax.dot_general(x1_ref[...], wb_ref[...], dn,
                           preferred_element_type=jnp.float32)
    acc += lax.dot_general(x2_ref[...], wc_ref[...], dn,
                           preferred_element_type=jnp.float32)
    acc += b_ref[...]
    o_ref[...] = jnp.maximum(acc, 0.2 * acc)


def _pointwise(x0f, x1f, x2f, w3, b3, blk=512):
    m, c0 = x0f.shape
    c1 = x1f.shape[1]
    c2 = x2f.shape[1]
    o = w3.shape[0]
    wa = w3[:, :c0]
    wb = w3[:, c0:c0 + c1]
    wc = w3[:, c0 + c1:]
    return pl.pallas_call(
        _pw_body,
        grid=(m // blk,),
        in_specs=[
            pl.BlockSpec((blk, c0), lambda i: (i, 0)),
            pl.BlockSpec((blk, c1), lambda i: (i, 0)),
            pl.BlockSpec((blk, c2), lambda i: (i, 0)),
            pl.BlockSpec((o, c0), lambda i: (0, 0)),
            pl.BlockSpec((o, c1), lambda i: (0, 0)),
            pl.BlockSpec((o, c2), lambda i: (0, 0)),
            pl.BlockSpec((1, o), lambda i: (0, 0)),
        ],
        out_specs=pl.BlockSpec((blk, o), lambda i: (i, 0)),
        out_shape=jax.ShapeDtypeStruct((m, o), jnp.float32),
    )(x0f, x1f, x2f, wa, wb, wc, b3.reshape(1, o))


def _edgeconv(xt, w, bias):
    b, n, c = xt.shape
    idx = _topk(xt)
    idx10 = idx[:, :, :_K].reshape(-1, 80)
    xf = xt.reshape(b * n, c)
    p, q = _proj(xf, w[:, :c], w[:, c:], bias)
    xo = _gather_max(idx10, q, p)
    return xo.reshape(b, n, -1)


def kernel(features, W1, b1, W2, b2, W3, b3):
    b, f, n = features.shape
    x0t_all = jnp.transpose(features, (0, 2, 1))
    outs = []
    gp = 2
    for bi in range(0, b, gp):
        x0t = lax.slice_in_dim(x0t_all, bi, bi + gp, axis=0)
        x0f = x0t.reshape(gp * n, f)
        idx1 = _topk(x0t)[:, :, :_K].reshape(-1, 80)
        e3 = _gather_sub(idx1, x0f)
        x1f = _edge_mlp(e3, x0f, W1, b1)
        x2f = _edgeconv(x1f.reshape(gp, n, f), W2, b2).reshape(gp * n, 2 * f)
        outs.append(_pointwise(x0f, x1f, x2f, W3, b3).reshape(gp, n, f))
    out = jnp.concatenate(outs, axis=0)
    return jnp.transpose(out, (0, 2, 1))


# speedup vs baseline: 18.6918x; 1.0183x over previous
import functools

import jax
import jax.numpy as jnp
from jax import lax
from jax.experimental import pallas as pl
from jax.experimental.pallas import tpu as pltpu
from jax.experimental.pallas import tpu_sc as plsc

_K = 10
_KPAD = 16
_NEG = -3.0e38

_NC = 2
_NS = 16
_NW = _NC * _NS


def _topk_body(xr_ref, xc_ref, idx_ref):
    xr = xr_ref[0]
    xc = xc_ref[0]
    blk, _ = xr.shape
    n = xc.shape[0]
    sq = jnp.sum(xc * xc, axis=1)
    s = 2.0 * lax.dot_general(xr, xc, (((1,), (1,)), ((), ())),
                              preferred_element_type=jnp.float32)
    s = s - sq[None, :]
    fio = lax.broadcasted_iota(jnp.int32, (blk, n), 1).astype(jnp.float32)
    cols = []
    for _ in range(_K):
        m = jnp.max(s, axis=1, keepdims=True)
        hit = s == m
        am = jnp.min(jnp.where(hit, fio, 3.0e38), axis=1, keepdims=True)
        cols.append(am)
        s = jnp.where(hit, _NEG, s)
    self_idx = (pl.program_id(1) * blk
                + lax.broadcasted_iota(jnp.int32, (blk, 1), 0))
    base = pl.program_id(0) * n
    topf = jnp.concatenate(cols, axis=1).astype(jnp.int32)
    pad = jnp.concatenate([self_idx] * (_KPAD - _K), axis=1)
    idx_ref[0] = jnp.concatenate([topf, pad], axis=1) + base


def _topk(xt, blk=512):
    b, n, c = xt.shape
    return pl.pallas_call(
        _topk_body,
        grid=(b, n // blk),
        in_specs=[
            pl.BlockSpec((1, blk, c), lambda bi, i: (bi, i, 0)),
            pl.BlockSpec((1, n, c), lambda bi, i: (bi, 0, 0)),
        ],
        out_specs=pl.BlockSpec((1, blk, _KPAD), lambda bi, i: (bi, i, 0)),
        out_shape=jax.ShapeDtypeStruct((b, n, _KPAD), jnp.int32),
    )(xt, xt)


def _proj_body(x_ref, wa_ref, wb_ref, b_ref, p_ref, q_ref):
    x = x_ref[...]
    wb = wb_ref[...]
    wd = wa_ref[...] - wb
    dn = (((1,), (1,)), ((), ()))
    p_ref[...] = (lax.dot_general(x, wd, dn, preferred_element_type=jnp.float32)
                  + b_ref[...])
    q_ref[...] = lax.dot_general(x, wb, dn, preferred_element_type=jnp.float32)


def _proj(xf, wa, wb, bias, blk=512):
    m, c = xf.shape
    o = wa.shape[0]
    return pl.pallas_call(
        _proj_body,
        grid=(m // blk,),
        in_specs=[
            pl.BlockSpec((blk, c), lambda i: (i, 0)),
            pl.BlockSpec((o, c), lambda i: (0, 0)),
            pl.BlockSpec((o, c), lambda i: (0, 0)),
            pl.BlockSpec((1, o), lambda i: (0, 0)),
        ],
        out_specs=[
            pl.BlockSpec((blk, o), lambda i: (i, 0)),
            pl.BlockSpec((blk, o), lambda i: (i, 0)),
        ],
        out_shape=[
            jax.ShapeDtypeStruct((m, o), jnp.float32),
            jax.ShapeDtypeStruct((m, o), jnp.float32),
        ],
    )(xf, wa, wb, bias.reshape(1, o))


def _gather_sub(idx2d, xf):
    m, c = xf.shape
    per_w = m // _NW
    g = 16
    t = per_w // g
    nsl = (g * _K) // 80
    mesh = plsc.VectorSubcoreMesh(core_axis_name="c", subcore_axis_name="s",
                                  num_cores=_NC, num_subcores=_NS)

    @functools.partial(
        pl.kernel,
        out_type=jax.ShapeDtypeStruct((_K, m, c), jnp.float32),
        mesh=mesh,
        scratch_types=[
            pltpu.VMEM((2, nsl, 80), jnp.int32),
            pltpu.VMEM((2, g * _K, c), jnp.float32),
            pltpu.VMEM((2, g, c), jnp.float32),
            pltpu.VMEM((2, _K, g, c), jnp.float32),
            pltpu.SemaphoreType.DMA,
            pltpu.SemaphoreType.DMA,
            pltpu.SemaphoreType.DMA,
            pltpu.SemaphoreType.DMA,
            pltpu.SemaphoreType.DMA,
            pltpu.SemaphoreType.DMA,
        ],
    )
    def kern(idx_hbm, x_hbm, e_hbm, idx_v, rows_v, xi_v, o3_v,
             si0, si1, sg0, sg1, so0, so1):
        wid = lax.axis_index("c") * _NS + lax.axis_index("s")
        si = (si0, si1)
        sg = (sg0, sg1)
        so = (so0, so1)

        def idx_row(ch):
            return wid * (per_w * _K // 80) + ch * nsl

        def pt_base(ch):
            return wid * per_w + ch * g

        def fire_idx(ch, p):
            pltpu.async_copy(idx_hbm.at[pl.ds(idx_row(ch), nsl)],
                             idx_v.at[p], si[p])

        def wait_idx(p):
            pltpu.make_async_copy(idx_hbm.at[pl.ds(0, nsl)],
                                  idx_v.at[p], si[p]).wait()

        def fire_gathers(ch, p):
            for j in range(nsl):
                pltpu.async_copy(x_hbm.at[idx_v.at[p, j]],
                                 rows_v.at[p, pl.ds(j * 80, 80)], sg[p])
            pltpu.async_copy(x_hbm.at[pl.ds(pt_base(ch), g)], xi_v.at[p], sg[p])

        def wait_gathers(p):
            for j in range(nsl):
                pltpu.make_async_copy(x_hbm.at[idx_v.at[p, j]],
                                      rows_v.at[p, pl.ds(j * 80, 80)],
                                      sg[p]).wait()
            pltpu.make_async_copy(x_hbm.at[pl.ds(0, g)], xi_v.at[p],
                                  sg[p]).wait()

        def fire_outs(ch, p):
            for r in range(_K):
                pltpu.async_copy(o3_v.at[p, r],
                                 e_hbm.at[r, pl.ds(pt_base(ch), g)], so[p])

        def wait_outs(p):
            for r in range(_K):
                pltpu.make_async_copy(o3_v.at[p, r],
                                      e_hbm.at[r, pl.ds(0, g)], so[p]).wait()

        def compute(p):
            def cbody(gg, carry):
                for cc in range(c // 16):
                    sl = pl.ds(cc * 16, 16)
                    xiv = xi_v[p, gg, sl]
                    for r in range(_K):
                        o3_v[p, r, gg, sl] = rows_v[p, gg * _K + r, sl] - xiv
                return carry
            lax.fori_loop(0, g, cbody, 0)

        def chunk_step(ch, p, drain_outs):
            q = 1 - p
            if drain_outs:
                wait_outs(p)
            wait_gathers(p)
            nxt = jnp.minimum(ch + 1, t - 1)
            wait_idx(q)
            fire_gathers(nxt, q)
            fire_idx(jnp.minimum(ch + 2, t - 1), p)
            compute(p)
            fire_outs(ch, p)

        fire_idx(0, 0)
        wait_idx(0)
        fire_gathers(0, 0)
        fire_idx(1, 1)
        chunk_step(0, 0, False)
        chunk_step(1, 1, False)

        def body(u, carry):
            chunk_step(2 * u, 0, True)
            chunk_step(2 * u + 1, 1, True)
            return carry

        lax.fori_loop(1, t // 2, body, 0)

        wait_outs(0)
        wait_outs(1)
        wait_gathers(0)
        wait_idx(1)

    return kern(idx2d, xf)


def _edge_mlp_body(e_ref, xi_ref, w_ref, b_ref, o_ref):
    e3 = e_ref[...]
    xi = xi_ref[...]
    k, blk, c = e3.shape
    xib = jnp.broadcast_to(xi[None], (k, blk, c))
    edge = jnp.concatenate([xib, e3], axis=2).reshape(k * blk, 2 * c)
    h = lax.dot_general(edge, w_ref[...], (((1,), (1,)), ((), ())),
                        preferred_element_type=jnp.float32)
    h = h + b_ref[...]
    h = jnp.where(h >= 0, h, 0.2 * h)
    o_ref[...] = jnp.max(h.reshape(k, blk, -1), axis=0)


def _edge_mlp(e3, xf, w, bias, blk=128):
    m, c = xf.shape
    o = w.shape[0]
    return pl.pallas_call(
        _edge_mlp_body,
        grid=(m // blk,),
        in_specs=[
            pl.BlockSpec((_K, blk, c), lambda i: (0, i, 0)),
            pl.BlockSpec((blk, c), lambda i: (i, 0)),
            pl.BlockSpec((o, 2 * c), lambda i: (0, 0)),
            pl.BlockSpec((1, o), lambda i: (0, 0)),
        ],
        out_specs=pl.BlockSpec((blk, o), lambda i: (i, 0)),
        out_shape=jax.ShapeDtypeStruct((m, o), jnp.float32),
    )(e3, xf, w, bias.reshape(1, o))


def _gather_max(idx2d, q, p):
    m, c = q.shape
    per_w = m // _NW
    g = 16
    t = per_w // g
    nsl = (g * _K) // 80
    mesh = plsc.VectorSubcoreMesh(core_axis_name="c", subcore_axis_name="s",
                                  num_cores=_NC, num_subcores=_NS)

    @functools.partial(
        pl.kernel,
        out_type=jax.ShapeDtypeStruct((m, c), jnp.float32),
        mesh=mesh,
        scratch_types=[
            pltpu.VMEM((2, nsl, 80), jnp.int32),
            pltpu.VMEM((2, g * _K, c), jnp.float32),
            pltpu.VMEM((2, g, c), jnp.float32),
            pltpu.VMEM((2, g, c), jnp.float32),
            pltpu.SemaphoreType.DMA,
            pltpu.SemaphoreType.DMA,
            pltpu.SemaphoreType.DMA,
            pltpu.SemaphoreType.DMA,
            pltpu.SemaphoreType.DMA,
            pltpu.SemaphoreType.DMA,
        ],
    )
    def kern(idx_hbm, q_hbm, p_hbm, out_hbm, idx_v, rows_v, p_v, o_v,
             si0, si1, sg0, sg1, so0, so1):
        wid = lax.axis_index("c") * _NS + lax.axis_index("s")
        si = (si0, si1)
        sg = (sg0, sg1)
        so = (so0, so1)

        def idx_row(ch):
            return wid * (per_w * _K // 80) + ch * nsl

        def pt_base(ch):
            return wid * per_w + ch * g

        def fire_idx(ch, pp):
            pltpu.async_copy(idx_hbm.at[pl.ds(idx_row(ch), nsl)],
                             idx_v.at[pp], si[pp])

        def wait_idx(pp):
            pltpu.make_async_copy(idx_hbm.at[pl.ds(0, nsl)],
                                  idx_v.at[pp], si[pp]).wait()

        def fire_gathers(ch, pp):
            for j in range(nsl):
                pltpu.async_copy(q_hbm.at[idx_v.at[pp, j]],
                                 rows_v.at[pp, pl.ds(j * 80, 80)], sg[pp])
            pltpu.async_copy(p_hbm.at[pl.ds(pt_base(ch), g)], p_v.at[pp],
                             sg[pp])

        def wait_gathers(pp):
            for j in range(nsl):
                pltpu.make_async_copy(q_hbm.at[idx_v.at[pp, j]],
                                      rows_v.at[pp, pl.ds(j * 80, 80)],
                                      sg[pp]).wait()
            pltpu.make_async_copy(p_hbm.at[pl.ds(0, g)], p_v.at[pp],
                                  sg[pp]).wait()

        def fire_outs(ch, pp):
            pltpu.async_copy(o_v.at[pp], out_hbm.at[pl.ds(pt_base(ch), g)],
                             so[pp])

        def wait_outs(pp):
            pltpu.make_async_copy(o_v.at[pp], out_hbm.at[pl.ds(0, g)],
                                  so[pp]).wait()

        def compute(pp):
            def cbody(gg, carry):
                for cc in range(c // 16):
                    sl = pl.ds(cc * 16, 16)
                    acc = rows_v[pp, gg * _K, sl]
                    for r in range(1, _K):
                        acc = jnp.maximum(acc, rows_v[pp, gg * _K + r, sl])
                    h = p_v[pp, gg, sl] + acc
                    o_v[pp, gg, sl] = jnp.maximum(h, 0.2 * h)
                return carry
            lax.fori_loop(0, g, cbody, 0)

        def chunk_step(ch, pp, drain_outs):
            qq = 1 - pp
            if drain_outs:
                wait_outs(pp)
            wait_gathers(pp)
            nxt = jnp.minimum(ch + 1, t - 1)
            wait_idx(qq)
            fire_gathers(nxt, qq)
            fire_idx(jnp.minimum(ch + 2, t - 1), pp)
            compute(pp)
            fire_outs(ch, pp)

        fire_idx(0, 0)
        wait_idx(0)
        fire_gathers(0, 0)
        fire_idx(1, 1)
        chunk_step(0, 0, False)
        chunk_step(1, 1, False)

        def body(u, carry):
            chunk_step(2 * u, 0, True)
            chunk_step(2 * u + 1, 1, True)
            return carry

        lax.fori_loop(1, t // 2, body, 0)

        wait_outs(0)
        wait_outs(1)
        wait_gathers(0)
        wait_idx(1)

    return kern(idx2d, q, p)


def _pw_body(x0_ref, x1_ref, x2_ref, wa_ref, wb_ref, wc_ref, b_ref, o_ref):
    dn = (((1,), (1,)), ((), ()))
    acc = lax.dot_general(x0_ref[...], wa_ref[...], dn,
                          preferred_element_type=jnp.float32)
    acc += lax.dot_general(x1_ref[...], wb_ref[...], dn,
                           preferred_element_type=jnp.float32)
    acc += lax.dot_general(x2_ref[...], wc_ref[...], dn,
                           preferred_element_type=jnp.float32)
    acc += b_ref[...]
    o_ref[...] = jnp.maximum(acc, 0.2 * acc)


def _pointwise(x0f, x1f, x2f, w3, b3, blk=512):
    m, c0 = x0f.shape
    c1 = x1f.shape[1]
    c2 = x2f.shape[1]
    o = w3.shape[0]
    wa = w3[:, :c0]
    wb = w3[:, c0:c0 + c1]
    wc = w3[:, c0 + c1:]
    return pl.pallas_call(
        _pw_body,
        grid=(m // blk,),
        in_specs=[
            pl.BlockSpec((blk, c0), lambda i: (i, 0)),
            pl.BlockSpec((blk, c1), lambda i: (i, 0)),
            pl.BlockSpec((blk, c2), lambda i: (i, 0)),
            pl.BlockSpec((o, c0), lambda i: (0, 0)),
            pl.BlockSpec((o, c1), lambda i: (0, 0)),
            pl.BlockSpec((o, c2), lambda i: (0, 0)),
            pl.BlockSpec((1, o), lambda i: (0, 0)),
        ],
        out_specs=pl.BlockSpec((blk, o), lambda i: (i, 0)),
        out_shape=jax.ShapeDtypeStruct((m, o), jnp.float32),
    )(x0f, x1f, x2f, wa, wb, wc, b3.reshape(1, o))


def _edgeconv(xt, w, bias):
    b, n, c = xt.shape
    idx = _topk(xt)
    idx10 = idx[:, :, :_K].reshape(-1, 80)
    xf = xt.reshape(b * n, c)
    p, q = _proj(xf, w[:, :c], w[:, c:], bias)
    xo = _gather_max(idx10, q, p)
    return xo.reshape(b, n, -1)


def kernel(features, W1, b1, W2, b2, W3, b3):
    b, f, n = features.shape
    x0t_all = jnp.transpose(features, (0, 2, 1))
    outs = []
    gp = 2
    for bi in range(0, b, gp):
        x0t = lax.slice_in_dim(x0t_all, bi, bi + gp, axis=0)
        x0f = x0t.reshape(gp * n, f)
        idx1 = _topk(x0t)[:, :, :_K].reshape(-1, 80)
        e3 = _gather_sub(idx1, x0f)
        x1f = _edge_mlp(e3, x0f, W1, b1)
        x2f = _edgeconv(x1f.reshape(gp, n, f), W2, b2).reshape(gp * n, 2 * f)
        outs.append(_pointwise(x0f, x1f, x2f, W3, b3).reshape(gp, n, f))
    out = jnp.concatenate(outs, axis=0)
    return jnp.transpose(out, (0, 2, 1))


# speedup vs baseline: 18.6953x; 1.0002x over previous
import functools

import jax
import jax.numpy as jnp
from jax import lax
from jax.experimental import pallas as pl
from jax.experimental.pallas import tpu as pltpu
from jax.experimental.pallas import tpu_sc as plsc

_K = 10
_KPAD = 16
_NEG = -3.0e38

_NC = 2
_NS = 16
_NW = _NC * _NS


def _topk_body(xr_ref, xc_ref, idx_ref):
    xr = xr_ref[0]
    xc = xc_ref[0]
    blk, _ = xr.shape
    n = xc.shape[0]
    sq = jnp.sum(xc * xc, axis=1)
    s = 2.0 * lax.dot_general(xr, xc, (((1,), (1,)), ((), ())),
                              preferred_element_type=jnp.float32)
    s = s - sq[None, :]
    fio = lax.broadcasted_iota(jnp.int32, (blk, n), 1).astype(jnp.float32)
    cols = []
    for _ in range(_K):
        m = jnp.max(s, axis=1, keepdims=True)
        hit = s == m
        am = jnp.min(jnp.where(hit, fio, 3.0e38), axis=1, keepdims=True)
        cols.append(am)
        s = jnp.where(hit, _NEG, s)
    self_idx = (pl.program_id(1) * blk
                + lax.broadcasted_iota(jnp.int32, (blk, 1), 0))
    base = pl.program_id(0) * n
    topf = jnp.concatenate(cols, axis=1).astype(jnp.int32)
    pad = jnp.concatenate([self_idx] * (_KPAD - _K), axis=1)
    idx_ref[0] = jnp.concatenate([topf, pad], axis=1) + base


def _topk(xt, blk=512):
    b, n, c = xt.shape
    return pl.pallas_call(
        _topk_body,
        grid=(b, n // blk),
        in_specs=[
            pl.BlockSpec((1, blk, c), lambda bi, i: (bi, i, 0)),
            pl.BlockSpec((1, n, c), lambda bi, i: (bi, 0, 0)),
        ],
        out_specs=pl.BlockSpec((1, blk, _KPAD), lambda bi, i: (bi, i, 0)),
        out_shape=jax.ShapeDtypeStruct((b, n, _KPAD), jnp.int32),
    )(xt, xt)


def _proj_body(x_ref, wa_ref, wb_ref, b_ref, p_ref, q_ref):
    x = x_ref[...]
    wb = wb_ref[...]
    wd = wa_ref[...] - wb
    dn = (((1,), (1,)), ((), ()))
    p_ref[...] = (lax.dot_general(x, wd, dn, preferred_element_type=jnp.float32)
                  + b_ref[...])
    q_ref[...] = lax.dot_general(x, wb, dn, preferred_element_type=jnp.float32)


def _proj(xf, wa, wb, bias, blk=512):
    m, c = xf.shape
    o = wa.shape[0]
    return pl.pallas_call(
        _proj_body,
        grid=(m // blk,),
        in_specs=[
            pl.BlockSpec((blk, c), lambda i: (i, 0)),
            pl.BlockSpec((o, c), lambda i: (0, 0)),
            pl.BlockSpec((o, c), lambda i: (0, 0)),
            pl.BlockSpec((1, o), lambda i: (0, 0)),
        ],
        out_specs=[
            pl.BlockSpec((blk, o), lambda i: (i, 0)),
            pl.BlockSpec((blk, o), lambda i: (i, 0)),
        ],
        out_shape=[
            jax.ShapeDtypeStruct((m, o), jnp.float32),
            jax.ShapeDtypeStruct((m, o), jnp.float32),
        ],
    )(xf, wa, wb, bias.reshape(1, o))


def _gather_sub(idx2d, xf):
    m, c = xf.shape
    per_w = m // _NW
    g = 16
    t = per_w // g
    nsl = (g * _K) // 80
    mesh = plsc.VectorSubcoreMesh(core_axis_name="c", subcore_axis_name="s",
                                  num_cores=_NC, num_subcores=_NS)

    @functools.partial(
        pl.kernel,
        out_type=jax.ShapeDtypeStruct((_K, m, c), jnp.float32),
        mesh=mesh,
        scratch_types=[
            pltpu.VMEM((2, nsl, 80), jnp.int32),
            pltpu.VMEM((2, g * _K, c), jnp.float32),
            pltpu.VMEM((2, g, c), jnp.float32),
            pltpu.VMEM((2, _K, g, c), jnp.float32),
            pltpu.SemaphoreType.DMA,
            pltpu.SemaphoreType.DMA,
            pltpu.SemaphoreType.DMA,
            pltpu.SemaphoreType.DMA,
            pltpu.SemaphoreType.DMA,
            pltpu.SemaphoreType.DMA,
        ],
    )
    def kern(idx_hbm, x_hbm, e_hbm, idx_v, rows_v, xi_v, o3_v,
             si0, si1, sg0, sg1, so0, so1):
        wid = lax.axis_index("c") * _NS + lax.axis_index("s")
        si = (si0, si1)
        sg = (sg0, sg1)
        so = (so0, so1)

        def idx_row(ch):
            return wid * (per_w * _K // 80) + ch * nsl

        def pt_base(ch):
            return wid * per_w + ch * g

        def fire_idx(ch, p):
            pltpu.async_copy(idx_hbm.at[pl.ds(idx_row(ch), nsl)],
                             idx_v.at[p], si[p])

        def wait_idx(p):
            pltpu.make_async_copy(idx_hbm.at[pl.ds(0, nsl)],
                                  idx_v.at[p], si[p]).wait()

        def fire_gathers(ch, p):
            for j in range(nsl):
                pltpu.async_copy(x_hbm.at[idx_v.at[p, j]],
                                 rows_v.at[p, pl.ds(j * 80, 80)], sg[p])
            pltpu.async_copy(x_hbm.at[pl.ds(pt_base(ch), g)], xi_v.at[p], sg[p])

        def wait_gathers(p):
            for j in range(nsl):
                pltpu.make_async_copy(x_hbm.at[idx_v.at[p, j]],
                                      rows_v.at[p, pl.ds(j * 80, 80)],
                                      sg[p]).wait()
            pltpu.make_async_copy(x_hbm.at[pl.ds(0, g)], xi_v.at[p],
                                  sg[p]).wait()

        def fire_outs(ch, p):
            for r in range(_K):
                pltpu.async_copy(o3_v.at[p, r],
                                 e_hbm.at[r, pl.ds(pt_base(ch), g)], so[p])

        def wait_outs(p):
            for r in range(_K):
                pltpu.make_async_copy(o3_v.at[p, r],
                                      e_hbm.at[r, pl.ds(0, g)], so[p]).wait()

        def compute(p):
            def cbody(gg, carry):
                for cc in range(c // 16):
                    sl = pl.ds(cc * 16, 16)
                    xiv = xi_v[p, gg, sl]
                    for r in range(_K):
                        o3_v[p, r, gg, sl] = rows_v[p, gg * _K + r, sl] - xiv
                return carry
            lax.fori_loop(0, g, cbody, 0)

        def chunk_step(ch, p, drain_outs):
            q = 1 - p
            if drain_outs:
                wait_outs(p)
            wait_gathers(p)
            nxt = jnp.minimum(ch + 1, t - 1)
            wait_idx(q)
            fire_gathers(nxt, q)
            fire_idx(jnp.minimum(ch + 2, t - 1), p)
            compute(p)
            fire_outs(ch, p)

        fire_idx(0, 0)
        wait_idx(0)
        fire_gathers(0, 0)
        fire_idx(1, 1)
        chunk_step(0, 0, False)
        chunk_step(1, 1, False)

        def body(u, carry):
            chunk_step(2 * u, 0, True)
            chunk_step(2 * u + 1, 1, True)
            return carry

        lax.fori_loop(1, t // 2, body, 0)

        wait_outs(0)
        wait_outs(1)
        wait_gathers(0)
        wait_idx(1)

    return kern(idx2d, xf)


def _edge_mlp_body(e_ref, xi_ref, w_ref, b_ref, o_ref):
    e3 = e_ref[...]
    xi = xi_ref[...]
    k, blk, c = e3.shape
    xib = jnp.broadcast_to(xi[None], (k, blk, c))
    edge = jnp.concatenate([xib, e3], axis=2).reshape(k * blk, 2 * c)
    h = lax.dot_general(edge, w_ref[...], (((1,), (1,)), ((), ())),
                        preferred_element_type=jnp.float32)
    h = h + b_ref[...]
    h = jnp.where(h >= 0, h, 0.2 * h)
    o_ref[...] = jnp.max(h.reshape(k, blk, -1), axis=0)


def _edge_mlp(e3, xf, w, bias, blk=128):
    m, c = xf.shape
    o = w.shape[0]
    return pl.pallas_call(
        _edge_mlp_body,
        grid=(m // blk,),
        in_specs=[
            pl.BlockSpec((_K, blk, c), lambda i: (0, i, 0)),
            pl.BlockSpec((blk, c), lambda i: (i, 0)),
            pl.BlockSpec((o, 2 * c), lambda i: (0, 0)),
            pl.BlockSpec((1, o), lambda i: (0, 0)),
        ],
        out_specs=pl.BlockSpec((blk, o), lambda i: (i, 0)),
        out_shape=jax.ShapeDtypeStruct((m, o), jnp.float32),
    )(e3, xf, w, bias.reshape(1, o))


def _gather_max(idx2d, q, p):
    m, c = q.shape
    per_w = m // _NW
    g = 16
    t = per_w // g
    nsl = (g * _K) // 80
    mesh = plsc.VectorSubcoreMesh(core_axis_name="c", subcore_axis_name="s",
                                  num_cores=_NC, num_subcores=_NS)

    @functools.partial(
        pl.kernel,
        out_type=jax.ShapeDtypeStruct((m, c), jnp.float32),
        mesh=mesh,
        scratch_types=[
            pltpu.VMEM((2, nsl, 80), jnp.int32),
            pltpu.VMEM((2, g * _K, c), jnp.float32),
            pltpu.VMEM((2, g, c), jnp.float32),
            pltpu.VMEM((2, g, c), jnp.float32),
            pltpu.SemaphoreType.DMA,
            pltpu.SemaphoreType.DMA,
            pltpu.SemaphoreType.DMA,
            pltpu.SemaphoreType.DMA,
            pltpu.SemaphoreType.DMA,
            pltpu.SemaphoreType.DMA,
        ],
    )
    def kern(idx_hbm, q_hbm, p_hbm, out_hbm, idx_v, rows_v, p_v, o_v,
             si0, si1, sg0, sg1, so0, so1):
        wid = lax.axis_index("c") * _NS + lax.axis_index("s")
        si = (si0, si1)
        sg = (sg0, sg1)
        so = (so0, so1)

        def idx_row(ch):
            return wid * (per_w * _K // 80) + ch * nsl

        def pt_base(ch):
            return wid * per_w + ch * g

        def fire_idx(ch, pp):
            pltpu.async_copy(idx_hbm.at[pl.ds(idx_row(ch), nsl)],
                             idx_v.at[pp], si[pp])

        def wait_idx(pp):
            pltpu.make_async_copy(idx_hbm.at[pl.ds(0, nsl)],
                                  idx_v.at[pp], si[pp]).wait()

        def fire_gathers(ch, pp):
            for j in range(nsl):
                pltpu.async_copy(q_hbm.at[idx_v.at[pp, j]],
                                 rows_v.at[pp, pl.ds(j * 80, 80)], sg[pp])
            pltpu.async_copy(p_hbm.at[pl.ds(pt_base(ch), g)], p_v.at[pp],
                             sg[pp])

        def wait_gathers(pp):
            for j in range(nsl):
                pltpu.make_async_copy(q_hbm.at[idx_v.at[pp, j]],
                                      rows_v.at[pp, pl.ds(j * 80, 80)],
                                      sg[pp]).wait()
            pltpu.make_async_copy(p_hbm.at[pl.ds(0, g)], p_v.at[pp],
                                  sg[pp]).wait()

        def fire_outs(ch, pp):
            pltpu.async_copy(o_v.at[pp], out_hbm.at[pl.ds(pt_base(ch), g)],
                             so[pp])

        def wait_outs(pp):
            pltpu.make_async_copy(o_v.at[pp], out_hbm.at[pl.ds(0, g)],
                                  so[pp]).wait()

        def compute(pp):
            def cbody(gg, carry):
                for cc in range(c // 16):
                    sl = pl.ds(cc * 16, 16)
                    acc = rows_v[pp, gg * _K, sl]
                    for r in range(1, _K):
                        acc = jnp.maximum(acc, rows_v[pp, gg * _K + r, sl])
                    h = p_v[pp, gg, sl] + acc
                    o_v[pp, gg, sl] = jnp.maximum(h, 0.2 * h)
                return carry
            lax.fori_loop(0, g, cbody, 0)

        def chunk_step(ch, pp, drain_outs):
            qq = 1 - pp
            if drain_outs:
                wait_outs(pp)
            wait_gathers(pp)
            nxt = jnp.minimum(ch + 1, t - 1)
            wait_idx(qq)
            fire_gathers(nxt, qq)
            fire_idx(jnp.minimum(ch + 2, t - 1), pp)
            compute(pp)
            fire_outs(ch, pp)

        fire_idx(0, 0)
        wait_idx(0)
        fire_gathers(0, 0)
        fire_idx(1, 1)
        chunk_step(0, 0, False)
        chunk_step(1, 1, False)

        def body(u, carry):
            chunk_step(2 * u, 0, True)
            chunk_step(2 * u + 1, 1, True)
            return carry

        lax.fori_loop(1, t // 2, body, 0)

        wait_outs(0)
        wait_outs(1)
        wait_gathers(0)
        wait_idx(1)

    return kern(idx2d, q, p)


def _pw_body(x0_ref, x1_ref, x2_ref, wa_ref, wb_ref, wc_ref, b_ref, o_ref):
    dn = (((1,), (1,)), ((), ()))
    acc = lax.dot_general(x0_ref[...], wa_ref[...], dn,
                          preferred_element_type=jnp.float32)
    acc += lax.dot_general(x1_ref[...], wb_ref[...], dn,
                           preferred_element_type=jnp.float32)
    acc += lax.dot_general(x2_ref[...], wc_ref[...], dn,
                           preferred_element_type=jnp.float32)
    acc += b_ref[...]
    o_ref[...] = jnp.maximum(acc, 0.2 * acc)


def _pointwise(x0f, x1f, x2f, w3, b3, blk=512):
    m, c0 = x0f.shape
    c1 = x1f.shape[1]
    c2 = x2f.shape[1]
    o = w3.shape[0]
    wa = w3[:, :c0]
    wb = w3[:, c0:c0 + c1]
    wc = w3[:, c0 + c1:]
    return pl.pallas_call(
        _pw_body,
        grid=(m // blk,),
        in_specs=[
            pl.BlockSpec((blk, c0), lambda i: (i, 0)),
            pl.BlockSpec((blk, c1), lambda i: (i, 0)),
            pl.BlockSpec((blk, c2), lambda i: (i, 0)),
            pl.BlockSpec((o, c0), lambda i: (0, 0)),
            pl.BlockSpec((o, c1), lambda i: (0, 0)),
            pl.BlockSpec((o, c2), lambda i: (0, 0)),
            pl.BlockSpec((1, o), lambda i: (0, 0)),
        ],
        out_specs=pl.BlockSpec((blk, o), lambda i: (i, 0)),
        out_shape=jax.ShapeDtypeStruct((m, o), jnp.float32),
    )(x0f, x1f, x2f, wa, wb, wc, b3.reshape(1, o))


def _edgeconv(xt, w, bias):
    b, n, c = xt.shape
    idx = _topk(xt)
    idx10 = idx[:, :, :_K].reshape(-1, 80)
    xf = xt.reshape(b * n, c)
    p, q = _proj(xf, w[:, :c], w[:, c:], bias)
    xo = _gather_max(idx10, q, p)
    return xo.reshape(b, n, -1)


def kernel(features, W1, b1, W2, b2, W3, b3):
    b, f, n = features.shape
    x0t_all = jnp.transpose(features, (0, 2, 1))
    gp = 2
    grps = [lax.slice_in_dim(x0t_all, bi, bi + gp, axis=0)
            for bi in range(0, b, gp)]
    x0fs = [g.reshape(gp * n, f) for g in grps]
    idx1s = [_topk(g)[:, :, :_K].reshape(-1, 80) for g in grps]
    e3s = [_gather_sub(i, x) for i, x in zip(idx1s, x0fs)]
    x1fs = [_edge_mlp(e, x, W1, b1) for e, x in zip(e3s, x0fs)]
    x2fs = [_edgeconv(x1.reshape(gp, n, f), W2, b2).reshape(gp * n, 2 * f)
            for x1 in x1fs]
    outs = [_pointwise(x0, x1, x2, W3, b3).reshape(gp, n, f)
            for x0, x1, x2 in zip(x0fs, x1fs, x2fs)]
    out = jnp.concatenate(outs, axis=0)
    return jnp.transpose(out, (0, 2, 1))
